# baseline (speedup 1.0000x reference)
import functools

import jax
import jax.numpy as jnp
from jax import lax
from jax.experimental import pallas as pl
from jax.experimental.pallas import tpu as pltpu
from jax.experimental.pallas import tpu_sc as plsc

_NU = 25000
_NI = 25000
_N = _NU + _NI
_E = 800000
_K = 64
_BLK = 1000

_HALF = 25000
_ACC = 25088
_ROWS_PER_TILE = _ACC // 16
_EROW = 128
_GRP = 2
_EPAD = 819200
_NGRP = _EPAD // (_EROW * _GRP)
_WB = 1000


_RPT = _EPAD // _EROW // 16
_IB = 8
_NB = _RPT // _IB


def _seg_body(z, srcm, dstm, zz, out, sidx, dloc, rows, acc,
              isem0, isem1, gsem0, gsem1, ssem0, ssem1):
    c = lax.axis_index("c")
    s = lax.axis_index("s")
    base = c * _HALF
    isem = (isem0, isem1)
    gsem = (gsem0, gsem1)
    ssem = (ssem0, ssem1)
    tile_r0 = s * _RPT

    pltpu.sync_copy(zz.at[pl.ds(s * _ROWS_PER_TILE, _ROWS_PER_TILE)],
                    acc.at[pl.ds(s * _ROWS_PER_TILE, _ROWS_PER_TILE)])
    plsc.subcore_barrier()

    iota = lax.iota(jnp.int32, 16)

    def remap(p):
        for r in range(_IB):
            for i in range(_EROW // 16):
                d = dloc[p, r, pl.ds(i * 16, 16)]
                loc = d - base
                m = (loc >= 0) & (loc < _HALF)
                dloc[p, r, pl.ds(i * 16, 16)] = jnp.where(m, loc, _HALF + iota)

    def issue_idx(p, blk):
        r0 = tile_r0 + blk * _IB
        pltpu.async_copy(srcm.at[pl.ds(r0, _IB)], sidx.at[p], isem[p])
        pltpu.async_copy(dstm.at[pl.ds(r0, _IB)], dloc.at[p], isem[p])

    def wait_idx(p):
        pltpu.make_async_copy(srcm.at[pl.ds(0, _IB)], sidx.at[p], isem[p]).wait()
        pltpu.make_async_copy(dstm.at[pl.ds(0, _IB)], dloc.at[p], isem[p]).wait()

    def issue_gather(p, r, q):
        pltpu.async_copy(z.at[sidx.at[p, r]], rows.at[q], gsem[q])

    def wait_gather(q):
        pltpu.make_async_copy(z.at[sidx.at[0, 0]], rows.at[q], gsem[q]).wait()

    pltpu.sync_copy(srcm.at[pl.ds(tile_r0, _IB)], sidx.at[0])
    pltpu.sync_copy(dstm.at[pl.ds(tile_r0, _IB)], dloc.at[0])
    remap(0)
    issue_idx(1, 1)
    issue_gather(0, 0, 0)

    def loop(i, carry):
        for p in (0, 1):
            x = 2 * i + p
            pn = p ^ 1

            @pl.when(x + 1 < _NB)
            def _():
                wait_idx(pn)
                remap(pn)
            for r in range(_IB):
                q = r & 1
                g = x * _IB + r

                wait_gather(q)
                pltpu.async_copy(rows.at[q], acc.at[dloc.at[p, r]], ssem[q],
                                 add=True)

                @pl.when(g > 0)
                def _():
                    pltpu.make_async_copy(
                        rows.at[q ^ 1], acc.at[dloc.at[p, r]],
                        ssem[q ^ 1]).wait()

                @pl.when(g + 1 < _RPT)
                def _():
                    if r < _IB - 1:
                        issue_gather(p, r + 1, q ^ 1)
                    else:
                        issue_gather(pn, 0, q ^ 1)

            @pl.when(x + 2 < _NB)
            def _():
                issue_idx(p, x + 2)
        return carry

    lax.fori_loop(0, _NB // 2, loop, 0)
    pltpu.make_async_copy(rows.at[1], acc.at[dloc.at[1, _IB - 1]],
                          ssem[1]).wait()
    plsc.subcore_barrier()

    for k2 in range(( _HALF // _WB + 15) // 16):
        g = s + 16 * k2

        @pl.when(g < _HALF // _WB)
        def _():
            pltpu.sync_copy(acc.at[pl.ds(g * _WB, _WB)],
                            out.at[pl.ds(base + g * _WB, _WB)])


@jax.jit
def _segment_sum_sc(z, src2d, dst2d, zz):
    mesh = plsc.VectorSubcoreMesh(core_axis_name="c", subcore_axis_name="s")
    return pl.kernel(
        _seg_body,
        out_type=jax.ShapeDtypeStruct((_N, _K), jnp.float32),
        mesh=mesh,
        scratch_types=[
            pltpu.VMEM((2, _IB, _EROW), jnp.int32),
            pltpu.VMEM((2, _IB, _EROW), jnp.int32),
            pltpu.VMEM((2, _EROW, _K), jnp.float32),
            pltpu.VMEM_SHARED((_ACC, _K), jnp.float32),
            pltpu.SemaphoreType.DMA,
            pltpu.SemaphoreType.DMA,
            pltpu.SemaphoreType.DMA,
            pltpu.SemaphoreType.DMA,
            pltpu.SemaphoreType.DMA,
            pltpu.SemaphoreType.DMA,
        ],
        compiler_params=pltpu.CompilerParams(use_tc_tiling_on_sc=False),
    )(z, src2d, dst2d, zz)


def _dotT(a, b):
    return lax.dot_general(a, b, (((1,), (1,)), ((), ())),
                           preferred_element_type=jnp.float32)


def _normalize(x):
    nrm = jnp.sqrt(jnp.sum(x * x, axis=1, keepdims=True))
    return x / jnp.maximum(nrm, 1e-12)


def _prep_items_body(fv, wv, bv, ft, wt, bt, ov, ot):
    ov[...] = _normalize(_dotT(fv[...], wv[...]) + bv[...])
    ot[...] = _normalize(_dotT(ft[...], wt[...]) + bt[...])


def _prep_users_body(gv, gt, ov, ot):
    ov[...] = _normalize(gv[...])
    ot[...] = _normalize(gt[...])


def _layer_core(x, pw, gw, lw, bias, ego):
    weff = _dotT(pw, gw)
    z = jnp.dot(x, weff, preferred_element_type=jnp.float32)
    xh = _dotT(x, lw) + bias + ego
    return z, xh


def _layer0_body(xv, pwv, gwv, lwv, bv, xt, pwt, gwt, lwt, bt, ego,
                 zv, xhv, zt, xht):
    zv[...], xhv[...] = _layer_core(xv[...], pwv[...], gwv[...], lwv[...],
                                    bv[...], ego[...])
    zt[...], xht[...] = _layer_core(xt[...], pwt[...], gwt[...], lwt[...],
                                    bt[...], ego[...])


def _layer1_body(sv, xpv, pwv, gwv, lwv, bv, st, xpt, pwt, gwt, lwt, bt, ego,
                 zv, xhv, zt, xht):
    zv[...], xhv[...] = _layer_core(sv[...] + xpv[...], pwv[...], gwv[...],
                                    lwv[...], bv[...], ego[...])
    zt[...], xht[...] = _layer_core(st[...] + xpt[...], pwt[...], gwt[...],
                                    lwt[...], bt[...], ego[...])


def _final_body(a, b, c, d, o):
    o[...] = a[...] + b[...] + c[...] + d[...]


def _row_spec(blk, k):
    return pl.BlockSpec((blk, k), lambda i: (i, 0))


def _full_spec(r, k):
    return pl.BlockSpec((r, k), lambda i: (0, 0))


def _prep_items(fv, wv, bv, ft, wt, bt):
    grid = _NI // _BLK
    out = jax.ShapeDtypeStruct((_NI, _K), jnp.float32)
    return pl.pallas_call(
        _prep_items_body,
        grid=(grid,),
        in_specs=[_row_spec(_BLK, 128), _full_spec(_K, 128), _full_spec(1, _K),
                  _row_spec(_BLK, 128), _full_spec(_K, 128), _full_spec(1, _K)],
        out_specs=[_row_spec(_BLK, _K)] * 2,
        out_shape=[out, out],
    )(fv, wv, bv, ft, wt, bt)


def _prep_users(gv, gt):
    grid = _NU // _BLK
    out = jax.ShapeDtypeStruct((_NU, _K), jnp.float32)
    return pl.pallas_call(
        _prep_users_body,
        grid=(grid,),
        in_specs=[_row_spec(_BLK, _K)] * 2,
        out_specs=[_row_spec(_BLK, _K)] * 2,
        out_shape=[out, out],
    )(gv, gt)


def _layer0(xv, pwv, gwv, lwv, bv, xt, pwt, gwt, lwt, bt, ego):
    grid = _N // _BLK
    out = jax.ShapeDtypeStruct((_N, _K), jnp.float32)
    w = _full_spec(_K, _K)
    b = _full_spec(1, _K)
    r = _row_spec(_BLK, _K)
    return pl.pallas_call(
        _layer0_body,
        grid=(grid,),
        in_specs=[r, w, w, w, b, r, w, w, w, b, r],
        out_specs=[r, r, r, r],
        out_shape=[out, out, out, out],
    )(xv, pwv, gwv, lwv, bv, xt, pwt, gwt, lwt, bt, ego)


def _layer1(sv, xpv, pwv, gwv, lwv, bv, st, xpt, pwt, gwt, lwt, bt, ego):
    grid = _N // _BLK
    out = jax.ShapeDtypeStruct((_N, _K), jnp.float32)
    w = _full_spec(_K, _K)
    b = _full_spec(1, _K)
    r = _row_spec(_BLK, _K)
    return pl.pallas_call(
        _layer1_body,
        grid=(grid,),
        in_specs=[r, r, w, w, w, b, r, r, w, w, w, b, r],
        out_specs=[r, r, r, r],
        out_shape=[out, out, out, out],
    )(sv, xpv, pwv, gwv, lwv, bv, st, xpt, pwt, gwt, lwt, bt, ego)


def _final(a, b, c, d):
    grid = _N // _BLK
    r = _row_spec(_BLK, _K)
    return pl.pallas_call(
        _final_body,
        grid=(grid,),
        in_specs=[r, r, r, r],
        out_specs=r,
        out_shape=jax.ShapeDtypeStruct((_N, _K), jnp.float32),
    )(a, b, c, d)


def kernel(edge_index, Gu, Gi, feat_visual, Gum_visual, proj_W_visual, proj_b_visual, prop_W_visual_0, lin_W_visual_0, lin_b_visual_0, g_W_visual_0, g_b_visual_0, prop_W_visual_1, lin_W_visual_1, lin_b_visual_1, g_W_visual_1, g_b_visual_1, feat_textual, Gum_textual, proj_W_textual, proj_b_textual, prop_W_textual_0, lin_W_textual_0, lin_b_textual_0, g_W_textual_0, g_b_textual_0, prop_W_textual_1, lin_W_textual_1, lin_b_textual_1, g_W_textual_1, g_b_textual_1):
    npad = _EPAD - _E
    pad_src = jnp.arange(npad, dtype=jnp.int32) & 16383
    pad_dst = jnp.full((npad,), _N, jnp.int32) + (jnp.arange(npad, dtype=jnp.int32) & 15)
    src2d = jnp.concatenate([edge_index[0], pad_src]).reshape(_EPAD // _EROW, _EROW)
    dst2d = jnp.concatenate([edge_index[1], pad_dst]).reshape(_EPAD // _EROW, _EROW)
    zz = jnp.zeros((_ACC, _K), jnp.float32)
    ego = jnp.concatenate([Gu, Gi], axis=0)

    bias0_v = (lin_b_visual_0 + g_b_visual_0).reshape(1, _K)
    bias1_v = (lin_b_visual_1 + g_b_visual_1).reshape(1, _K)
    bias0_t = (lin_b_textual_0 + g_b_textual_0).reshape(1, _K)
    bias1_t = (lin_b_textual_1 + g_b_textual_1).reshape(1, _K)

    xi_v, xi_t = _prep_items(feat_visual, proj_W_visual,
                             proj_b_visual.reshape(1, _K),
                             feat_textual, proj_W_textual,
                             proj_b_textual.reshape(1, _K))
    xu_v, xu_t = _prep_users(Gum_visual, Gum_textual)
    x0_v = jnp.concatenate([xu_v, xi_v], axis=0)
    x0_t = jnp.concatenate([xu_t, xi_t], axis=0)

    z0_v, xh0_v, z0_t, xh0_t = _layer0(
        x0_v, prop_W_visual_0, g_W_visual_0, lin_W_visual_0, bias0_v,
        x0_t, prop_W_textual_0, g_W_textual_0, lin_W_textual_0, bias0_t, ego)

    s0_v = _segment_sum_sc(z0_v, src2d, dst2d, zz)
    s0_t = _segment_sum_sc(z0_t, src2d, dst2d, zz)

    z1_v, xh1_v, z1_t, xh1_t = _layer1(
        s0_v, xh0_v, prop_W_visual_1, g_W_visual_1, lin_W_visual_1, bias1_v,
        s0_t, xh0_t, prop_W_textual_1, g_W_textual_1, lin_W_textual_1, bias1_t,
        ego)

    s1_v = _segment_sum_sc(z1_v, src2d, dst2d, zz)
    s1_t = _segment_sum_sc(z1_t, src2d, dst2d, zz)

    x_all = _final(s1_v, xh1_v, s1_t, xh1_t)
    return x_all[:_NU], x_all[_NU:]


# speedup vs baseline: 1.1793x; 1.1793x over previous
import functools

import jax
import jax.numpy as jnp
from jax import lax
from jax.experimental import pallas as pl
from jax.experimental.pallas import tpu as pltpu
from jax.experimental.pallas import tpu_sc as plsc

_NU = 25000
_NI = 25000
_N = _NU + _NI
_E = 800000
_K = 64
_BLK = 1000

_HALF = 25000
_ACC = 25088
_ROWS_PER_TILE = _ACC // 16
_EROW = 128
_GRP = 2
_EPAD = 819200
_NGRP = _EPAD // (_EROW * _GRP)
_WB = 1000


_RPT = _EPAD // _EROW // 16
_IB = 8
_NB = _RPT // _IB


def _seg_body(z, srcm, dstm, zz, out, sidx, dloc, rows, acc,
              isem0, isem1, gsem0, gsem1, ssem0, ssem1):
    c = lax.axis_index("c")
    s = lax.axis_index("s")
    base = c * _HALF
    isem = (isem0, isem1)
    gsem = (gsem0, gsem1)
    ssem = (ssem0, ssem1)
    tile_r0 = s * _RPT

    pltpu.sync_copy(zz.at[pl.ds(s * _ROWS_PER_TILE, _ROWS_PER_TILE)],
                    acc.at[pl.ds(s * _ROWS_PER_TILE, _ROWS_PER_TILE)])
    plsc.subcore_barrier()

    iota = lax.iota(jnp.int32, 16)

    def remap(p):
        for r in range(_IB):
            for i in range(_EROW // 16):
                d = dloc[p, r, pl.ds(i * 16, 16)]
                loc = d - base
                m = (loc >= 0) & (loc < _HALF)
                dloc[p, r, pl.ds(i * 16, 16)] = jnp.where(m, loc, _HALF + iota)

    def issue_idx(p, blk):
        r0 = tile_r0 + blk * _IB
        pltpu.async_copy(srcm.at[pl.ds(r0, _IB)], sidx.at[p], isem[p])
        pltpu.async_copy(dstm.at[pl.ds(r0, _IB)], dloc.at[p], isem[p])

    def wait_idx(p):
        pltpu.make_async_copy(srcm.at[pl.ds(0, _IB)], sidx.at[p], isem[p]).wait()
        pltpu.make_async_copy(dstm.at[pl.ds(0, _IB)], dloc.at[p], isem[p]).wait()

    def issue_gather(p, r, q):
        pltpu.async_copy(z.at[sidx.at[p, r]], rows.at[q], gsem[q])

    def wait_gather(q):
        pltpu.make_async_copy(z.at[sidx.at[0, 0]], rows.at[q], gsem[q]).wait()

    pltpu.sync_copy(srcm.at[pl.ds(tile_r0, _IB)], sidx.at[0])
    pltpu.sync_copy(dstm.at[pl.ds(tile_r0, _IB)], dloc.at[0])
    remap(0)
    issue_idx(1, 1)
    issue_gather(0, 0, 0)

    def loop(i, carry):
        for p in (0, 1):
            x = 2 * i + p
            pn = p ^ 1

            @pl.when(x + 1 < _NB)
            def _():
                wait_idx(pn)
                remap(pn)
            for r in range(_IB):
                q = r & 1
                g = x * _IB + r

                @pl.when(g + 1 < _RPT)
                def _():
                    if r < _IB - 1:
                        issue_gather(p, r + 1, q ^ 1)
                    else:
                        issue_gather(pn, 0, q ^ 1)
                wait_gather(q)
                pltpu.sync_copy(rows.at[q], acc.at[dloc.at[p, r]], add=True)

            @pl.when(x + 2 < _NB)
            def _():
                issue_idx(p, x + 2)
        return carry

    lax.fori_loop(0, _NB // 2, loop, 0)
    plsc.subcore_barrier()

    for k2 in range(( _HALF // _WB + 15) // 16):
        g = s + 16 * k2

        @pl.when(g < _HALF // _WB)
        def _():
            pltpu.sync_copy(acc.at[pl.ds(g * _WB, _WB)],
                            out.at[pl.ds(base + g * _WB, _WB)])


@jax.jit
def _segment_sum_sc(z, src2d, dst2d, zz):
    mesh = plsc.VectorSubcoreMesh(core_axis_name="c", subcore_axis_name="s")
    return pl.kernel(
        _seg_body,
        out_type=jax.ShapeDtypeStruct((_N, _K), jnp.float32),
        mesh=mesh,
        scratch_types=[
            pltpu.VMEM((2, _IB, _EROW), jnp.int32),
            pltpu.VMEM((2, _IB, _EROW), jnp.int32),
            pltpu.VMEM((2, _EROW, _K), jnp.float32),
            pltpu.VMEM_SHARED((_ACC, _K), jnp.float32),
            pltpu.SemaphoreType.DMA,
            pltpu.SemaphoreType.DMA,
            pltpu.SemaphoreType.DMA,
            pltpu.SemaphoreType.DMA,
            pltpu.SemaphoreType.DMA,
            pltpu.SemaphoreType.DMA,
        ],
        compiler_params=pltpu.CompilerParams(use_tc_tiling_on_sc=False),
    )(z, src2d, dst2d, zz)


def _dotT(a, b):
    return lax.dot_general(a, b, (((1,), (1,)), ((), ())),
                           preferred_element_type=jnp.float32)


def _normalize(x):
    nrm = jnp.sqrt(jnp.sum(x * x, axis=1, keepdims=True))
    return x / jnp.maximum(nrm, 1e-12)


def _prep_items_body(fv, wv, bv, ft, wt, bt, ov, ot):
    ov[...] = _normalize(_dotT(fv[...], wv[...]) + bv[...])
    ot[...] = _normalize(_dotT(ft[...], wt[...]) + bt[...])


def _prep_users_body(gv, gt, ov, ot):
    ov[...] = _normalize(gv[...])
    ot[...] = _normalize(gt[...])


def _layer_core(x, pw, gw, lw, bias, ego):
    weff = _dotT(pw, gw)
    z = jnp.dot(x, weff, preferred_element_type=jnp.float32)
    xh = _dotT(x, lw) + bias + ego
    return z, xh


def _layer0_body(xv, pwv, gwv, lwv, bv, xt, pwt, gwt, lwt, bt, ego,
                 zv, xhv, zt, xht):
    zv[...], xhv[...] = _layer_core(xv[...], pwv[...], gwv[...], lwv[...],
                                    bv[...], ego[...])
    zt[...], xht[...] = _layer_core(xt[...], pwt[...], gwt[...], lwt[...],
                                    bt[...], ego[...])


def _layer1_body(sv, xpv, pwv, gwv, lwv, bv, st, xpt, pwt, gwt, lwt, bt, ego,
                 zv, xhv, zt, xht):
    zv[...], xhv[...] = _layer_core(sv[...] + xpv[...], pwv[...], gwv[...],
                                    lwv[...], bv[...], ego[...])
    zt[...], xht[...] = _layer_core(st[...] + xpt[...], pwt[...], gwt[...],
                                    lwt[...], bt[...], ego[...])


def _final_body(a, b, c, d, o):
    o[...] = a[...] + b[...] + c[...] + d[...]


def _row_spec(blk, k):
    return pl.BlockSpec((blk, k), lambda i: (i, 0))


def _full_spec(r, k):
    return pl.BlockSpec((r, k), lambda i: (0, 0))


def _prep_items(fv, wv, bv, ft, wt, bt):
    grid = _NI // _BLK
    out = jax.ShapeDtypeStruct((_NI, _K), jnp.float32)
    return pl.pallas_call(
        _prep_items_body,
        grid=(grid,),
        in_specs=[_row_spec(_BLK, 128), _full_spec(_K, 128), _full_spec(1, _K),
                  _row_spec(_BLK, 128), _full_spec(_K, 128), _full_spec(1, _K)],
        out_specs=[_row_spec(_BLK, _K)] * 2,
        out_shape=[out, out],
    )(fv, wv, bv, ft, wt, bt)


def _prep_users(gv, gt):
    grid = _NU // _BLK
    out = jax.ShapeDtypeStruct((_NU, _K), jnp.float32)
    return pl.pallas_call(
        _prep_users_body,
        grid=(grid,),
        in_specs=[_row_spec(_BLK, _K)] * 2,
        out_specs=[_row_spec(_BLK, _K)] * 2,
        out_shape=[out, out],
    )(gv, gt)


def _layer0(xv, pwv, gwv, lwv, bv, xt, pwt, gwt, lwt, bt, ego):
    grid = _N // _BLK
    out = jax.ShapeDtypeStruct((_N, _K), jnp.float32)
    w = _full_spec(_K, _K)
    b = _full_spec(1, _K)
    r = _row_spec(_BLK, _K)
    return pl.pallas_call(
        _layer0_body,
        grid=(grid,),
        in_specs=[r, w, w, w, b, r, w, w, w, b, r],
        out_specs=[r, r, r, r],
        out_shape=[out, out, out, out],
    )(xv, pwv, gwv, lwv, bv, xt, pwt, gwt, lwt, bt, ego)


def _layer1(sv, xpv, pwv, gwv, lwv, bv, st, xpt, pwt, gwt, lwt, bt, ego):
    grid = _N // _BLK
    out = jax.ShapeDtypeStruct((_N, _K), jnp.float32)
    w = _full_spec(_K, _K)
    b = _full_spec(1, _K)
    r = _row_spec(_BLK, _K)
    return pl.pallas_call(
        _layer1_body,
        grid=(grid,),
        in_specs=[r, r, w, w, w, b, r, r, w, w, w, b, r],
        out_specs=[r, r, r, r],
        out_shape=[out, out, out, out],
    )(sv, xpv, pwv, gwv, lwv, bv, st, xpt, pwt, gwt, lwt, bt, ego)


def _final(a, b, c, d):
    grid = _N // _BLK
    r = _row_spec(_BLK, _K)
    return pl.pallas_call(
        _final_body,
        grid=(grid,),
        in_specs=[r, r, r, r],
        out_specs=r,
        out_shape=jax.ShapeDtypeStruct((_N, _K), jnp.float32),
    )(a, b, c, d)


def kernel(edge_index, Gu, Gi, feat_visual, Gum_visual, proj_W_visual, proj_b_visual, prop_W_visual_0, lin_W_visual_0, lin_b_visual_0, g_W_visual_0, g_b_visual_0, prop_W_visual_1, lin_W_visual_1, lin_b_visual_1, g_W_visual_1, g_b_visual_1, feat_textual, Gum_textual, proj_W_textual, proj_b_textual, prop_W_textual_0, lin_W_textual_0, lin_b_textual_0, g_W_textual_0, g_b_textual_0, prop_W_textual_1, lin_W_textual_1, lin_b_textual_1, g_W_textual_1, g_b_textual_1):
    npad = _EPAD - _E
    pad_src = jnp.arange(npad, dtype=jnp.int32) & 16383
    pad_dst = jnp.full((npad,), _N, jnp.int32) + (jnp.arange(npad, dtype=jnp.int32) & 15)
    src2d = jnp.concatenate([edge_index[0], pad_src]).reshape(_EPAD // _EROW, _EROW)
    dst2d = jnp.concatenate([edge_index[1], pad_dst]).reshape(_EPAD // _EROW, _EROW)
    zz = jnp.zeros((_ACC, _K), jnp.float32)
    ego = jnp.concatenate([Gu, Gi], axis=0)

    bias0_v = (lin_b_visual_0 + g_b_visual_0).reshape(1, _K)
    bias1_v = (lin_b_visual_1 + g_b_visual_1).reshape(1, _K)
    bias0_t = (lin_b_textual_0 + g_b_textual_0).reshape(1, _K)
    bias1_t = (lin_b_textual_1 + g_b_textual_1).reshape(1, _K)

    xi_v, xi_t = _prep_items(feat_visual, proj_W_visual,
                             proj_b_visual.reshape(1, _K),
                             feat_textual, proj_W_textual,
                             proj_b_textual.reshape(1, _K))
    xu_v, xu_t = _prep_users(Gum_visual, Gum_textual)
    x0_v = jnp.concatenate([xu_v, xi_v], axis=0)
    x0_t = jnp.concatenate([xu_t, xi_t], axis=0)

    z0_v, xh0_v, z0_t, xh0_t = _layer0(
        x0_v, prop_W_visual_0, g_W_visual_0, lin_W_visual_0, bias0_v,
        x0_t, prop_W_textual_0, g_W_textual_0, lin_W_textual_0, bias0_t, ego)

    s0_v = _segment_sum_sc(z0_v, src2d, dst2d, zz)
    s0_t = _segment_sum_sc(z0_t, src2d, dst2d, zz)

    z1_v, xh1_v, z1_t, xh1_t = _layer1(
        s0_v, xh0_v, prop_W_visual_1, g_W_visual_1, lin_W_visual_1, bias1_v,
        s0_t, xh0_t, prop_W_textual_1, g_W_textual_1, lin_W_textual_1, bias1_t,
        ego)

    s1_v = _segment_sum_sc(z1_v, src2d, dst2d, zz)
    s1_t = _segment_sum_sc(z1_t, src2d, dst2d, zz)

    x_all = _final(s1_v, xh1_v, s1_t, xh1_t)
    return x_all[:_NU], x_all[_NU:]


# speedup vs baseline: 1.6498x; 1.3990x over previous
import functools

import jax
import jax.numpy as jnp
from jax import lax
from jax.experimental import pallas as pl
from jax.experimental.pallas import tpu as pltpu
from jax.experimental.pallas import tpu_sc as plsc

_NU = 25000
_NI = 25000
_N = _NU + _NI
_E = 800000
_K = 64
_BLK = 1000

_HALF = 25000
_ACC = 25088
_ROWS_PER_TILE = _ACC // 16
_EROW = 128
_GRP = 2
_EPAD = 819200
_NGRP = _EPAD // (_EROW * _GRP)
_WB = 1000


_RPT = _EPAD // _EROW // 16
_IB = 8
_NB = _RPT // _IB

_NSTRIP = 32
_SROWS = _EPAD // _EROW // _NSTRIP
_FIB = 10
_PADU = 2048
_CAP = 28672
_CROWS = _CAP // _EROW


def _seg_body(z, srcm, dstm, zz, out, sidx, dloc, rows, acc,
              isem0, isem1, gsem0, gsem1, ssem0, ssem1):
    c = lax.axis_index("c")
    s = lax.axis_index("s")
    base = c * _HALF
    isem = (isem0, isem1)
    gsem = (gsem0, gsem1)
    ssem = (ssem0, ssem1)
    tile_r0 = s * _RPT

    pltpu.sync_copy(zz.at[pl.ds(s * _ROWS_PER_TILE, _ROWS_PER_TILE)],
                    acc.at[pl.ds(s * _ROWS_PER_TILE, _ROWS_PER_TILE)])
    plsc.subcore_barrier()

    iota = lax.iota(jnp.int32, 16)

    def remap(p):
        for r in range(_IB):
            for i in range(_EROW // 16):
                d = dloc[p, r, pl.ds(i * 16, 16)]
                loc = d - base
                m = (loc >= 0) & (loc < _HALF)
                dloc[p, r, pl.ds(i * 16, 16)] = jnp.where(m, loc, _HALF + iota)

    def issue_idx(p, blk):
        r0 = tile_r0 + blk * _IB
        pltpu.async_copy(srcm.at[pl.ds(r0, _IB)], sidx.at[p], isem[p])
        pltpu.async_copy(dstm.at[pl.ds(r0, _IB)], dloc.at[p], isem[p])

    def wait_idx(p):
        pltpu.make_async_copy(srcm.at[pl.ds(0, _IB)], sidx.at[p], isem[p]).wait()
        pltpu.make_async_copy(dstm.at[pl.ds(0, _IB)], dloc.at[p], isem[p]).wait()

    def issue_gather(p, r, q):
        pltpu.async_copy(z.at[sidx.at[p, r]], rows.at[q], gsem[q])

    def wait_gather(q):
        pltpu.make_async_copy(z.at[sidx.at[0, 0]], rows.at[q], gsem[q]).wait()

    pltpu.sync_copy(srcm.at[pl.ds(tile_r0, _IB)], sidx.at[0])
    pltpu.sync_copy(dstm.at[pl.ds(tile_r0, _IB)], dloc.at[0])
    remap(0)
    issue_idx(1, 1)
    issue_gather(0, 0, 0)

    def loop(i, carry):
        for p in (0, 1):
            x = 2 * i + p
            pn = p ^ 1

            @pl.when(x + 1 < _NB)
            def _():
                wait_idx(pn)
                remap(pn)
            for r in range(_IB):
                q = r & 1
                g = x * _IB + r

                @pl.when(g + 1 < _RPT)
                def _():
                    if r < _IB - 1:
                        issue_gather(p, r + 1, q ^ 1)
                    else:
                        issue_gather(pn, 0, q ^ 1)
                wait_gather(q)
                pltpu.sync_copy(rows.at[q], acc.at[dloc.at[p, r]], add=True)

            @pl.when(x + 2 < _NB)
            def _():
                issue_idx(p, x + 2)
        return carry

    lax.fori_loop(0, _NB // 2, loop, 0)
    plsc.subcore_barrier()

    for k2 in range(( _HALF // _WB + 15) // 16):
        g = s + 16 * k2

        @pl.when(g < _HALF // _WB)
        def _():
            pltpu.sync_copy(acc.at[pl.ds(g * _WB, _WB)],
                            out.at[pl.ds(base + g * _WB, _WB)])


def _filter_body(srcm, dstm, csrc, cdst, counts, isrc, idst,
                 bsrc0, bdst0, bsrc1, bdst1, cb, isem0, isem1):
    c = lax.axis_index("c")
    s = lax.axis_index("s")
    w = c * 16 + s
    strip0 = w * _SROWS
    isem = (isem0, isem1)
    iota = lax.iota(jnp.int32, 16)
    bufs = ((bsrc0, bdst0), (bsrc1, bdst1))

    def issue_idx(p, blk):
        r0 = strip0 + blk * _FIB
        pltpu.async_copy(srcm.at[pl.ds(r0, _FIB)], isrc.at[p], isem[p])
        pltpu.async_copy(dstm.at[pl.ds(r0, _FIB)], idst.at[p], isem[p])

    def wait_idx(p):
        pltpu.make_async_copy(srcm.at[pl.ds(0, _FIB)], isrc.at[p], isem[p]).wait()
        pltpu.make_async_copy(dstm.at[pl.ds(0, _FIB)], idst.at[p], isem[p]).wait()

    pltpu.sync_copy(srcm.at[pl.ds(strip0, _FIB)], isrc.at[0])
    pltpu.sync_copy(dstm.at[pl.ds(strip0, _FIB)], idst.at[0])
    issue_idx(1, 1)

    nblk = _SROWS // _FIB

    def loop(i, carry):
        n0, n1 = carry
        for p in (0, 1):
            x = 2 * i + p
            pn = p ^ 1

            @pl.when(x + 1 < nblk)
            def _():
                wait_idx(pn)
            for r in range(_FIB):
                for v in range(_EROW // 16):
                    sv = isrc[p, r, pl.ds(v * 16, 16)]
                    dv = idst[p, r, pl.ds(v * 16, 16)]
                    m0 = dv < _HALF
                    pref0 = plsc.cumsum(m0.astype(jnp.int32))
                    pos0 = n0 + pref0 - 1
                    plsc.store_scatter(bsrc0, [pos0], sv, mask=m0)
                    plsc.store_scatter(bdst0, [pos0], dv, mask=m0)
                    n0 = n0 + jnp.max(pref0)
                    loc = dv - _HALF
                    m1 = (loc >= 0) & (loc < _HALF)
                    pref1 = plsc.cumsum(m1.astype(jnp.int32))
                    pos1 = n1 + pref1 - 1
                    plsc.store_scatter(bsrc1, [pos1], sv, mask=m1)
                    plsc.store_scatter(bdst1, [pos1], loc, mask=m1)
                    n1 = n1 + jnp.max(pref1)

            @pl.when(x + 2 < nblk)
            def _():
                issue_idx(p, x + 2)
        return (n0, n1)

    n0, n1 = lax.fori_loop(0, nblk // 2, loop,
                           (jnp.int32(0), jnp.int32(0)))

    for h, n in ((0, n0), (1, n1)):
        bs, bd = bufs[h]
        for i in range(_PADU // 16):
            src_pad = (iota + 16 * i) & 16383
            bs[pl.ds(n + 16 * i, 16)] = src_pad
            bd[pl.ds(n + 16 * i, 16)] = _HALF + iota
        npairs = (n + _PADU - 1) >> 11
        cb[...] = jnp.broadcast_to(npairs, (16,)).astype(jnp.int32)
        pltpu.sync_copy(cb, counts.at[h, w])
        pltpu.sync_copy(bs, csrc.at[h, w])
        pltpu.sync_copy(bd, cdst.at[h, w])


@jax.jit
def _edge_partition_sc(src2d, dst2d):
    mesh = plsc.VectorSubcoreMesh(core_axis_name="c", subcore_axis_name="s")
    return pl.kernel(
        _filter_body,
        out_type=(
            jax.ShapeDtypeStruct((2, _NSTRIP, _CAP), jnp.int32),
            jax.ShapeDtypeStruct((2, _NSTRIP, _CAP), jnp.int32),
            jax.ShapeDtypeStruct((2, _NSTRIP, 16), jnp.int32),
        ),
        mesh=mesh,
        scratch_types=[
            pltpu.VMEM((2, _FIB, _EROW), jnp.int32),
            pltpu.VMEM((2, _FIB, _EROW), jnp.int32),
            pltpu.VMEM((_CAP,), jnp.int32),
            pltpu.VMEM((_CAP,), jnp.int32),
            pltpu.VMEM((_CAP,), jnp.int32),
            pltpu.VMEM((_CAP,), jnp.int32),
            pltpu.VMEM((16,), jnp.int32),
            pltpu.SemaphoreType.DMA,
            pltpu.SemaphoreType.DMA,
        ],
        compiler_params=pltpu.CompilerParams(use_tc_tiling_on_sc=False,
                                             needs_layout_passes=False),
    )(src2d, dst2d)


def _seg_compact_body(z, csrc, cdst, counts, zz, out, sidx, dloc, rows, cntv,
                      acc, isem0, isem1, gsem0, gsem1):
    c = lax.axis_index("c")
    s = lax.axis_index("s")
    base = c * _HALF
    isem = (isem0, isem1)
    gsem = (gsem0, gsem1)

    pltpu.sync_copy(zz.at[pl.ds(s * _ROWS_PER_TILE, _ROWS_PER_TILE)],
                    acc.at[pl.ds(s * _ROWS_PER_TILE, _ROWS_PER_TILE)])
    plsc.subcore_barrier()

    def issue_gather(p, r, q):
        pltpu.async_copy(z.at[sidx.at[p, r]], rows.at[q], gsem[q])

    def wait_gather(q):
        pltpu.make_async_copy(z.at[sidx.at[0, 0]], rows.at[q], gsem[q]).wait()

    for reg in (0, 1):
        w = 2 * s + reg
        pltpu.sync_copy(counts.at[c, w], cntv)
        npairs = jnp.max(cntv[...])
        nblocks = npairs * 2
        nrows = nblocks * _IB

        def issue_idx(p, blk):
            pltpu.async_copy(csrc.at[c, w, pl.ds(blk * _IB, _IB)],
                             sidx.at[p], isem[p])
            pltpu.async_copy(cdst.at[c, w, pl.ds(blk * _IB, _IB)],
                             dloc.at[p], isem[p])

        def wait_idx(p):
            pltpu.make_async_copy(csrc.at[c, w, pl.ds(0, _IB)],
                                  sidx.at[p], isem[p]).wait()
            pltpu.make_async_copy(cdst.at[c, w, pl.ds(0, _IB)],
                                  dloc.at[p], isem[p]).wait()

        pltpu.sync_copy(csrc.at[c, w, pl.ds(0, _IB)], sidx.at[0])
        pltpu.sync_copy(cdst.at[c, w, pl.ds(0, _IB)], dloc.at[0])

        @pl.when(nblocks > 1)
        def _():
            issue_idx(1, 1)
        issue_gather(0, 0, 0)

        def loop(bp, carry):
            for p in (0, 1):
                x = 2 * bp + p
                pn = p ^ 1

                @pl.when(x + 1 < nblocks)
                def _():
                    wait_idx(pn)
                for r in range(_IB):
                    q = r & 1
                    g = x * _IB + r

                    @pl.when(g + 1 < nrows)
                    def _():
                        if r < _IB - 1:
                            issue_gather(p, r + 1, q ^ 1)
                        else:
                            issue_gather(pn, 0, q ^ 1)
                    wait_gather(q)
                    pltpu.sync_copy(rows.at[q], acc.at[dloc.at[p, r]],
                                    add=True)

                @pl.when(x + 2 < nblocks)
                def _():
                    issue_idx(p, x + 2)
            return carry

        lax.fori_loop(0, npairs, loop, 0)

    plsc.subcore_barrier()

    for k2 in range(( _HALF // _WB + 15) // 16):
        g = s + 16 * k2

        @pl.when(g < _HALF // _WB)
        def _():
            pltpu.sync_copy(acc.at[pl.ds(g * _WB, _WB)],
                            out.at[pl.ds(base + g * _WB, _WB)])


@jax.jit
def _segment_sum_compact_sc(z, csrc, cdst, counts, zz):
    mesh = plsc.VectorSubcoreMesh(core_axis_name="c", subcore_axis_name="s")
    return pl.kernel(
        _seg_compact_body,
        out_type=jax.ShapeDtypeStruct((_N, _K), jnp.float32),
        mesh=mesh,
        scratch_types=[
            pltpu.VMEM((2, _IB, _EROW), jnp.int32),
            pltpu.VMEM((2, _IB, _EROW), jnp.int32),
            pltpu.VMEM((2, _EROW, _K), jnp.float32),
            pltpu.VMEM((16,), jnp.int32),
            pltpu.VMEM_SHARED((_ACC, _K), jnp.float32),
            pltpu.SemaphoreType.DMA,
            pltpu.SemaphoreType.DMA,
            pltpu.SemaphoreType.DMA,
            pltpu.SemaphoreType.DMA,
        ],
        compiler_params=pltpu.CompilerParams(use_tc_tiling_on_sc=False,
                                             needs_layout_passes=False),
    )(z, csrc, cdst, counts, zz)


@jax.jit
def _segment_sum_sc(z, src2d, dst2d, zz):
    mesh = plsc.VectorSubcoreMesh(core_axis_name="c", subcore_axis_name="s")
    return pl.kernel(
        _seg_body,
        out_type=jax.ShapeDtypeStruct((_N, _K), jnp.float32),
        mesh=mesh,
        scratch_types=[
            pltpu.VMEM((2, _IB, _EROW), jnp.int32),
            pltpu.VMEM((2, _IB, _EROW), jnp.int32),
            pltpu.VMEM((2, _EROW, _K), jnp.float32),
            pltpu.VMEM_SHARED((_ACC, _K), jnp.float32),
            pltpu.SemaphoreType.DMA,
            pltpu.SemaphoreType.DMA,
            pltpu.SemaphoreType.DMA,
            pltpu.SemaphoreType.DMA,
            pltpu.SemaphoreType.DMA,
            pltpu.SemaphoreType.DMA,
        ],
        compiler_params=pltpu.CompilerParams(use_tc_tiling_on_sc=False),
    )(z, src2d, dst2d, zz)


def _dotT(a, b):
    return lax.dot_general(a, b, (((1,), (1,)), ((), ())),
                           preferred_element_type=jnp.float32)


def _normalize(x):
    nrm = jnp.sqrt(jnp.sum(x * x, axis=1, keepdims=True))
    return x / jnp.maximum(nrm, 1e-12)


def _prep_items_body(fv, wv, bv, ft, wt, bt, ov, ot):
    ov[...] = _normalize(_dotT(fv[...], wv[...]) + bv[...])
    ot[...] = _normalize(_dotT(ft[...], wt[...]) + bt[...])


def _prep_users_body(gv, gt, ov, ot):
    ov[...] = _normalize(gv[...])
    ot[...] = _normalize(gt[...])


def _layer_core(x, pw, gw, lw, bias, ego):
    weff = _dotT(pw, gw)
    z = jnp.dot(x, weff, preferred_element_type=jnp.float32)
    xh = _dotT(x, lw) + bias + ego
    return z, xh


def _layer0_body(xv, pwv, gwv, lwv, bv, xt, pwt, gwt, lwt, bt, ego,
                 zv, xhv, zt, xht):
    zv[...], xhv[...] = _layer_core(xv[...], pwv[...], gwv[...], lwv[...],
                                    bv[...], ego[...])
    zt[...], xht[...] = _layer_core(xt[...], pwt[...], gwt[...], lwt[...],
                                    bt[...], ego[...])


def _layer1_body(sv, xpv, pwv, gwv, lwv, bv, st, xpt, pwt, gwt, lwt, bt, ego,
                 zv, xhv, zt, xht):
    zv[...], xhv[...] = _layer_core(sv[...] + xpv[...], pwv[...], gwv[...],
                                    lwv[...], bv[...], ego[...])
    zt[...], xht[...] = _layer_core(st[...] + xpt[...], pwt[...], gwt[...],
                                    lwt[...], bt[...], ego[...])


def _final_body(a, b, c, d, o):
    o[...] = a[...] + b[...] + c[...] + d[...]


def _row_spec(blk, k):
    return pl.BlockSpec((blk, k), lambda i: (i, 0))


def _full_spec(r, k):
    return pl.BlockSpec((r, k), lambda i: (0, 0))


def _prep_items(fv, wv, bv, ft, wt, bt):
    grid = _NI // _BLK
    out = jax.ShapeDtypeStruct((_NI, _K), jnp.float32)
    return pl.pallas_call(
        _prep_items_body,
        grid=(grid,),
        in_specs=[_row_spec(_BLK, 128), _full_spec(_K, 128), _full_spec(1, _K),
                  _row_spec(_BLK, 128), _full_spec(_K, 128), _full_spec(1, _K)],
        out_specs=[_row_spec(_BLK, _K)] * 2,
        out_shape=[out, out],
    )(fv, wv, bv, ft, wt, bt)


def _prep_users(gv, gt):
    grid = _NU // _BLK
    out = jax.ShapeDtypeStruct((_NU, _K), jnp.float32)
    return pl.pallas_call(
        _prep_users_body,
        grid=(grid,),
        in_specs=[_row_spec(_BLK, _K)] * 2,
        out_specs=[_row_spec(_BLK, _K)] * 2,
        out_shape=[out, out],
    )(gv, gt)


def _layer0(xv, pwv, gwv, lwv, bv, xt, pwt, gwt, lwt, bt, ego):
    grid = _N // _BLK
    out = jax.ShapeDtypeStruct((_N, _K), jnp.float32)
    w = _full_spec(_K, _K)
    b = _full_spec(1, _K)
    r = _row_spec(_BLK, _K)
    return pl.pallas_call(
        _layer0_body,
        grid=(grid,),
        in_specs=[r, w, w, w, b, r, w, w, w, b, r],
        out_specs=[r, r, r, r],
        out_shape=[out, out, out, out],
    )(xv, pwv, gwv, lwv, bv, xt, pwt, gwt, lwt, bt, ego)


def _layer1(sv, xpv, pwv, gwv, lwv, bv, st, xpt, pwt, gwt, lwt, bt, ego):
    grid = _N // _BLK
    out = jax.ShapeDtypeStruct((_N, _K), jnp.float32)
    w = _full_spec(_K, _K)
    b = _full_spec(1, _K)
    r = _row_spec(_BLK, _K)
    return pl.pallas_call(
        _layer1_body,
        grid=(grid,),
        in_specs=[r, r, w, w, w, b, r, r, w, w, w, b, r],
        out_specs=[r, r, r, r],
        out_shape=[out, out, out, out],
    )(sv, xpv, pwv, gwv, lwv, bv, st, xpt, pwt, gwt, lwt, bt, ego)


def _final(a, b, c, d):
    grid = _N // _BLK
    r = _row_spec(_BLK, _K)
    return pl.pallas_call(
        _final_body,
        grid=(grid,),
        in_specs=[r, r, r, r],
        out_specs=r,
        out_shape=jax.ShapeDtypeStruct((_N, _K), jnp.float32),
    )(a, b, c, d)


def kernel(edge_index, Gu, Gi, feat_visual, Gum_visual, proj_W_visual, proj_b_visual, prop_W_visual_0, lin_W_visual_0, lin_b_visual_0, g_W_visual_0, g_b_visual_0, prop_W_visual_1, lin_W_visual_1, lin_b_visual_1, g_W_visual_1, g_b_visual_1, feat_textual, Gum_textual, proj_W_textual, proj_b_textual, prop_W_textual_0, lin_W_textual_0, lin_b_textual_0, g_W_textual_0, g_b_textual_0, prop_W_textual_1, lin_W_textual_1, lin_b_textual_1, g_W_textual_1, g_b_textual_1):
    npad = _EPAD - _E
    pad_src = jnp.arange(npad, dtype=jnp.int32) & 16383
    pad_dst = jnp.full((npad,), _N, jnp.int32) + (jnp.arange(npad, dtype=jnp.int32) & 15)
    src2d = jnp.concatenate([edge_index[0], pad_src]).reshape(_EPAD // _EROW, _EROW)
    dst2d = jnp.concatenate([edge_index[1], pad_dst]).reshape(_EPAD // _EROW, _EROW)
    zz = jnp.zeros((_ACC, _K), jnp.float32)

    csrc, cdst, counts = _edge_partition_sc(src2d, dst2d)
    csrc = csrc.reshape(2, _NSTRIP, _CROWS, _EROW)
    cdst = cdst.reshape(2, _NSTRIP, _CROWS, _EROW)
    ego = jnp.concatenate([Gu, Gi], axis=0)

    bias0_v = (lin_b_visual_0 + g_b_visual_0).reshape(1, _K)
    bias1_v = (lin_b_visual_1 + g_b_visual_1).reshape(1, _K)
    bias0_t = (lin_b_textual_0 + g_b_textual_0).reshape(1, _K)
    bias1_t = (lin_b_textual_1 + g_b_textual_1).reshape(1, _K)

    xi_v, xi_t = _prep_items(feat_visual, proj_W_visual,
                             proj_b_visual.reshape(1, _K),
                             feat_textual, proj_W_textual,
                             proj_b_textual.reshape(1, _K))
    xu_v, xu_t = _prep_users(Gum_visual, Gum_textual)
    x0_v = jnp.concatenate([xu_v, xi_v], axis=0)
    x0_t = jnp.concatenate([xu_t, xi_t], axis=0)

    z0_v, xh0_v, z0_t, xh0_t = _layer0(
        x0_v, prop_W_visual_0, g_W_visual_0, lin_W_visual_0, bias0_v,
        x0_t, prop_W_textual_0, g_W_textual_0, lin_W_textual_0, bias0_t, ego)

    s0_v = _segment_sum_compact_sc(z0_v, csrc, cdst, counts, zz)
    s0_t = _segment_sum_compact_sc(z0_t, csrc, cdst, counts, zz)

    z1_v, xh1_v, z1_t, xh1_t = _layer1(
        s0_v, xh0_v, prop_W_visual_1, g_W_visual_1, lin_W_visual_1, bias1_v,
        s0_t, xh0_t, prop_W_textual_1, g_W_textual_1, lin_W_textual_1, bias1_t,
        ego)

    s1_v = _segment_sum_compact_sc(z1_v, csrc, cdst, counts, zz)
    s1_t = _segment_sum_compact_sc(z1_t, csrc, cdst, counts, zz)

    x_all = _final(s1_v, xh1_v, s1_t, xh1_t)
    return x_all[:_NU], x_all[_NU:]


# speedup vs baseline: 1.7648x; 1.0697x over previous
import functools

import jax
import jax.numpy as jnp
from jax import lax
from jax.experimental import pallas as pl
from jax.experimental.pallas import tpu as pltpu
from jax.experimental.pallas import tpu_sc as plsc

_NU = 25000
_NI = 25000
_N = _NU + _NI
_E = 800000
_K = 64
_BLK = 1000

_HALF = 25000
_ACC = 25088
_ROWS_PER_TILE = _ACC // 16
_EROW = 128
_GRP = 2
_EPAD = 819200
_NGRP = _EPAD // (_EROW * _GRP)
_WB = 1000


_RPT = _EPAD // _EROW // 16
_IB = 8
_NB = _RPT // _IB

_NSTRIP = 32
_SROWS = _EPAD // _EROW // _NSTRIP
_FIB = 10
_PADU = 2048
_CAP = 28672
_CROWS = _CAP // _EROW


def _seg_body(z, srcm, dstm, zz, out, sidx, dloc, rows, acc,
              isem0, isem1, gsem0, gsem1, ssem0, ssem1):
    c = lax.axis_index("c")
    s = lax.axis_index("s")
    base = c * _HALF
    isem = (isem0, isem1)
    gsem = (gsem0, gsem1)
    ssem = (ssem0, ssem1)
    tile_r0 = s * _RPT

    pltpu.sync_copy(zz.at[pl.ds(s * _ROWS_PER_TILE, _ROWS_PER_TILE)],
                    acc.at[pl.ds(s * _ROWS_PER_TILE, _ROWS_PER_TILE)])
    plsc.subcore_barrier()

    iota = lax.iota(jnp.int32, 16)

    def remap(p):
        for r in range(_IB):
            for i in range(_EROW // 16):
                d = dloc[p, r, pl.ds(i * 16, 16)]
                loc = d - base
                m = (loc >= 0) & (loc < _HALF)
                dloc[p, r, pl.ds(i * 16, 16)] = jnp.where(m, loc, _HALF + iota)

    def issue_idx(p, blk):
        r0 = tile_r0 + blk * _IB
        pltpu.async_copy(srcm.at[pl.ds(r0, _IB)], sidx.at[p], isem[p])
        pltpu.async_copy(dstm.at[pl.ds(r0, _IB)], dloc.at[p], isem[p])

    def wait_idx(p):
        pltpu.make_async_copy(srcm.at[pl.ds(0, _IB)], sidx.at[p], isem[p]).wait()
        pltpu.make_async_copy(dstm.at[pl.ds(0, _IB)], dloc.at[p], isem[p]).wait()

    def issue_gather(p, r, q):
        pltpu.async_copy(z.at[sidx.at[p, r]], rows.at[q], gsem[q])

    def wait_gather(q):
        pltpu.make_async_copy(z.at[sidx.at[0, 0]], rows.at[q], gsem[q]).wait()

    pltpu.sync_copy(srcm.at[pl.ds(tile_r0, _IB)], sidx.at[0])
    pltpu.sync_copy(dstm.at[pl.ds(tile_r0, _IB)], dloc.at[0])
    remap(0)
    issue_idx(1, 1)
    issue_gather(0, 0, 0)

    def loop(i, carry):
        for p in (0, 1):
            x = 2 * i + p
            pn = p ^ 1

            @pl.when(x + 1 < _NB)
            def _():
                wait_idx(pn)
                remap(pn)
            for r in range(_IB):
                q = r & 1
                g = x * _IB + r

                @pl.when(g + 1 < _RPT)
                def _():
                    if r < _IB - 1:
                        issue_gather(p, r + 1, q ^ 1)
                    else:
                        issue_gather(pn, 0, q ^ 1)
                wait_gather(q)
                pltpu.sync_copy(rows.at[q], acc.at[dloc.at[p, r]], add=True)

            @pl.when(x + 2 < _NB)
            def _():
                issue_idx(p, x + 2)
        return carry

    lax.fori_loop(0, _NB // 2, loop, 0)
    plsc.subcore_barrier()

    for k2 in range(( _HALF // _WB + 15) // 16):
        g = s + 16 * k2

        @pl.when(g < _HALF // _WB)
        def _():
            pltpu.sync_copy(acc.at[pl.ds(g * _WB, _WB)],
                            out.at[pl.ds(base + g * _WB, _WB)])


def _filter_body(srcm, dstm, csrc, cdst, counts, isrc, idst,
                 bsrc0, bdst0, bsrc1, bdst1, cb, isem0, isem1):
    c = lax.axis_index("c")
    s = lax.axis_index("s")
    w = c * 16 + s
    strip0 = w * _SROWS
    isem = (isem0, isem1)
    iota = lax.iota(jnp.int32, 16)
    bufs = ((bsrc0, bdst0), (bsrc1, bdst1))

    def issue_idx(p, blk):
        r0 = strip0 + blk * _FIB
        pltpu.async_copy(srcm.at[pl.ds(r0, _FIB)], isrc.at[p], isem[p])
        pltpu.async_copy(dstm.at[pl.ds(r0, _FIB)], idst.at[p], isem[p])

    def wait_idx(p):
        pltpu.make_async_copy(srcm.at[pl.ds(0, _FIB)], isrc.at[p], isem[p]).wait()
        pltpu.make_async_copy(dstm.at[pl.ds(0, _FIB)], idst.at[p], isem[p]).wait()

    pltpu.sync_copy(srcm.at[pl.ds(strip0, _FIB)], isrc.at[0])
    pltpu.sync_copy(dstm.at[pl.ds(strip0, _FIB)], idst.at[0])
    issue_idx(1, 1)

    nblk = _SROWS // _FIB

    def loop(i, carry):
        n0, n1 = carry
        for p in (0, 1):
            x = 2 * i + p
            pn = p ^ 1

            @pl.when(x + 1 < nblk)
            def _():
                wait_idx(pn)
            for r in range(_FIB):
                for v in range(_EROW // 16):
                    sv = isrc[p, r, pl.ds(v * 16, 16)]
                    dv = idst[p, r, pl.ds(v * 16, 16)]
                    m0 = dv < _HALF
                    pref0 = plsc.cumsum(m0.astype(jnp.int32))
                    pos0 = n0 + pref0 - 1
                    plsc.store_scatter(bsrc0, [pos0], sv, mask=m0)
                    plsc.store_scatter(bdst0, [pos0], dv, mask=m0)
                    n0 = n0 + jnp.max(pref0)
                    loc = dv - _HALF
                    m1 = (loc >= 0) & (loc < _HALF)
                    pref1 = plsc.cumsum(m1.astype(jnp.int32))
                    pos1 = n1 + pref1 - 1
                    plsc.store_scatter(bsrc1, [pos1], sv, mask=m1)
                    plsc.store_scatter(bdst1, [pos1], loc, mask=m1)
                    n1 = n1 + jnp.max(pref1)

            @pl.when(x + 2 < nblk)
            def _():
                issue_idx(p, x + 2)
        return (n0, n1)

    n0, n1 = lax.fori_loop(0, nblk // 2, loop,
                           (jnp.int32(0), jnp.int32(0)))

    for h, n in ((0, n0), (1, n1)):
        bs, bd = bufs[h]
        for i in range(_PADU // 16):
            src_pad = (iota + 16 * i) & 16383
            bs[pl.ds(n + 16 * i, 16)] = src_pad
            bd[pl.ds(n + 16 * i, 16)] = _HALF + iota
        npairs = (n + _PADU - 1) >> 11
        cb[...] = jnp.broadcast_to(npairs, (16,)).astype(jnp.int32)
        pltpu.sync_copy(cb, counts.at[h, w])
        pltpu.sync_copy(bs, csrc.at[h, w])
        pltpu.sync_copy(bd, cdst.at[h, w])


@jax.jit
def _edge_partition_sc(src2d, dst2d):
    mesh = plsc.VectorSubcoreMesh(core_axis_name="c", subcore_axis_name="s")
    return pl.kernel(
        _filter_body,
        out_type=(
            jax.ShapeDtypeStruct((2, _NSTRIP, _CAP), jnp.int32),
            jax.ShapeDtypeStruct((2, _NSTRIP, _CAP), jnp.int32),
            jax.ShapeDtypeStruct((2, _NSTRIP, 16), jnp.int32),
        ),
        mesh=mesh,
        scratch_types=[
            pltpu.VMEM((2, _FIB, _EROW), jnp.int32),
            pltpu.VMEM((2, _FIB, _EROW), jnp.int32),
            pltpu.VMEM((_CAP,), jnp.int32),
            pltpu.VMEM((_CAP,), jnp.int32),
            pltpu.VMEM((_CAP,), jnp.int32),
            pltpu.VMEM((_CAP,), jnp.int32),
            pltpu.VMEM((16,), jnp.int32),
            pltpu.SemaphoreType.DMA,
            pltpu.SemaphoreType.DMA,
        ],
        compiler_params=pltpu.CompilerParams(use_tc_tiling_on_sc=False,
                                             needs_layout_passes=False),
    )(src2d, dst2d)


def _seg_compact_body(z, csrc, cdst, counts, zz, out, sidx, dloc, rows, cntv,
                      acc, isem0, isem1, gsem0, gsem1):
    c = lax.axis_index("c")
    s = lax.axis_index("s")
    base = c * _HALF
    isem = (isem0, isem1)
    gsem = (gsem0, gsem1)

    pltpu.sync_copy(zz.at[pl.ds(s * _ROWS_PER_TILE, _ROWS_PER_TILE)],
                    acc.at[pl.ds(s * _ROWS_PER_TILE, _ROWS_PER_TILE)])
    plsc.subcore_barrier()

    def issue_gather(p, r, q):
        pltpu.async_copy(z.at[sidx.at[p, r]], rows.at[q], gsem[q])

    def wait_gather(q):
        pltpu.make_async_copy(z.at[sidx.at[0, 0]], rows.at[q], gsem[q]).wait()

    for reg in (0, 1):
        w = 2 * s + reg
        pltpu.sync_copy(counts.at[c, w], cntv)
        npairs = jnp.max(cntv[...])
        nblocks = npairs * 2
        nrows = nblocks * _IB

        def issue_idx(p, blk):
            pltpu.async_copy(csrc.at[c, w, pl.ds(blk * _IB, _IB)],
                             sidx.at[p], isem[p])
            pltpu.async_copy(cdst.at[c, w, pl.ds(blk * _IB, _IB)],
                             dloc.at[p], isem[p])

        def wait_idx(p):
            pltpu.make_async_copy(csrc.at[c, w, pl.ds(0, _IB)],
                                  sidx.at[p], isem[p]).wait()
            pltpu.make_async_copy(cdst.at[c, w, pl.ds(0, _IB)],
                                  dloc.at[p], isem[p]).wait()

        pltpu.sync_copy(csrc.at[c, w, pl.ds(0, _IB)], sidx.at[0])
        pltpu.sync_copy(cdst.at[c, w, pl.ds(0, _IB)], dloc.at[0])

        @pl.when(nblocks > 1)
        def _():
            issue_idx(1, 1)
        issue_gather(0, 0, 0)

        def loop(bp, carry):
            for p in (0, 1):
                x = 2 * bp + p
                pn = p ^ 1

                @pl.when(x + 1 < nblocks)
                def _():
                    wait_idx(pn)
                for r in range(_IB):
                    q = r & 1
                    g = x * _IB + r

                    @pl.when(g + 1 < nrows)
                    def _():
                        if r < _IB - 1:
                            issue_gather(p, r + 1, q ^ 1)
                        else:
                            issue_gather(pn, 0, q ^ 1)
                    wait_gather(q)
                    pltpu.sync_copy(rows.at[q], acc.at[dloc.at[p, r]],
                                    add=True)

                @pl.when(x + 2 < nblocks)
                def _():
                    issue_idx(p, x + 2)
            return carry

        lax.fori_loop(0, npairs, loop, 0)

    plsc.subcore_barrier()

    for k2 in range(( _HALF // _WB + 15) // 16):
        g = s + 16 * k2

        @pl.when(g < _HALF // _WB)
        def _():
            pltpu.sync_copy(acc.at[pl.ds(g * _WB, _WB)],
                            out.at[pl.ds(base + g * _WB, _WB)])


@jax.jit
def _segment_sum_compact_sc(z, csrc, cdst, counts, zz):
    mesh = plsc.VectorSubcoreMesh(core_axis_name="c", subcore_axis_name="s")
    return pl.kernel(
        _seg_compact_body,
        out_type=jax.ShapeDtypeStruct((_N, _K), jnp.float32),
        mesh=mesh,
        scratch_types=[
            pltpu.VMEM((2, _IB, _EROW), jnp.int32),
            pltpu.VMEM((2, _IB, _EROW), jnp.int32),
            pltpu.VMEM((2, _EROW, _K), jnp.float32),
            pltpu.VMEM((16,), jnp.int32),
            pltpu.VMEM_SHARED((_ACC, _K), jnp.float32),
            pltpu.SemaphoreType.DMA,
            pltpu.SemaphoreType.DMA,
            pltpu.SemaphoreType.DMA,
            pltpu.SemaphoreType.DMA,
        ],
        compiler_params=pltpu.CompilerParams(use_tc_tiling_on_sc=False,
                                             needs_layout_passes=False),
    )(z, csrc, cdst, counts, zz)


@jax.jit
def _segment_sum_sc(z, src2d, dst2d, zz):
    mesh = plsc.VectorSubcoreMesh(core_axis_name="c", subcore_axis_name="s")
    return pl.kernel(
        _seg_body,
        out_type=jax.ShapeDtypeStruct((_N, _K), jnp.float32),
        mesh=mesh,
        scratch_types=[
            pltpu.VMEM((2, _IB, _EROW), jnp.int32),
            pltpu.VMEM((2, _IB, _EROW), jnp.int32),
            pltpu.VMEM((2, _EROW, _K), jnp.float32),
            pltpu.VMEM_SHARED((_ACC, _K), jnp.float32),
            pltpu.SemaphoreType.DMA,
            pltpu.SemaphoreType.DMA,
            pltpu.SemaphoreType.DMA,
            pltpu.SemaphoreType.DMA,
            pltpu.SemaphoreType.DMA,
            pltpu.SemaphoreType.DMA,
        ],
        compiler_params=pltpu.CompilerParams(use_tc_tiling_on_sc=False),
    )(z, src2d, dst2d, zz)


def _dotT(a, b):
    return lax.dot_general(a, b, (((1,), (1,)), ((), ())),
                           preferred_element_type=jnp.float32)


def _normalize(x):
    nrm = jnp.sqrt(jnp.sum(x * x, axis=1, keepdims=True))
    return x / jnp.maximum(nrm, 1e-12)


def _layer_core(x, pw, gw, lw, bias, ego):
    weff = _dotT(pw, gw)
    z = jnp.dot(x, weff, preferred_element_type=jnp.float32)
    xh = _dotT(x, lw) + bias + ego
    return z, xh


def _stage0_body(gumv, fv, pjwv, pjbv, pwv, gwv, lwv, bv,
                 gumt, ft, pjwt, pjbt, pwt, gwt, lwt, bt, gu, gi,
                 zv, xhv, zt, xht):
    user = pl.program_id(0) < _NU // _BLK
    e = jnp.where(user, gu[...], gi[...])
    for (gum, f, pjw, pjb, pw, gw, lw, b, z, xh) in (
            (gumv, fv, pjwv, pjbv, pwv, gwv, lwv, bv, zv, xhv),
            (gumt, ft, pjwt, pjbt, pwt, gwt, lwt, bt, zt, xht)):
        xg = _normalize(gum[...])
        xp = _normalize(_dotT(f[...], pjw[...]) + pjb[...])
        x = jnp.where(user, xg, xp)
        z[...], xh[...] = _layer_core(x, pw[...], gw[...], lw[...], b[...], e)


def _layer1_body(sv, xpv, pwv, gwv, lwv, bv, st, xpt, pwt, gwt, lwt, bt,
                 gu, gi, zv, xhv, zt, xht):
    user = pl.program_id(0) < _NU // _BLK
    e = jnp.where(user, gu[...], gi[...])
    zv[...], xhv[...] = _layer_core(sv[...] + xpv[...], pwv[...], gwv[...],
                                    lwv[...], bv[...], e)
    zt[...], xht[...] = _layer_core(st[...] + xpt[...], pwt[...], gwt[...],
                                    lwt[...], bt[...], e)


def _final_body(a, b, c, d, o):
    o[...] = a[...] + b[...] + c[...] + d[...]


def _row_spec(blk, k):
    return pl.BlockSpec((blk, k), lambda i: (i, 0))


def _full_spec(r, k):
    return pl.BlockSpec((r, k), lambda i: (0, 0))


def _user_spec(blk, k):
    return pl.BlockSpec((blk, k), lambda i: (jnp.minimum(i, _NU // _BLK - 1), 0))


def _item_spec(blk, k):
    return pl.BlockSpec((blk, k), lambda i: (jnp.maximum(i - _NU // _BLK, 0), 0))


def _stage0(gumv, fv, pjwv, pjbv, pwv, gwv, lwv, bv,
            gumt, ft, pjwt, pjbt, pwt, gwt, lwt, bt, gu, gi):
    grid = _N // _BLK
    out = jax.ShapeDtypeStruct((_N, _K), jnp.float32)
    w = _full_spec(_K, _K)
    b = _full_spec(1, _K)
    r = _row_spec(_BLK, _K)
    mod = [_user_spec(_BLK, _K), _item_spec(_BLK, 128), _full_spec(_K, 128),
           b, w, w, w, b]
    return pl.pallas_call(
        _stage0_body,
        grid=(grid,),
        in_specs=mod + mod + [_user_spec(_BLK, _K), _item_spec(_BLK, _K)],
        out_specs=[r, r, r, r],
        out_shape=[out, out, out, out],
    )(gumv, fv, pjwv, pjbv, pwv, gwv, lwv, bv,
      gumt, ft, pjwt, pjbt, pwt, gwt, lwt, bt, gu, gi)


def _layer1(sv, xpv, pwv, gwv, lwv, bv, st, xpt, pwt, gwt, lwt, bt, gu, gi):
    grid = _N // _BLK
    out = jax.ShapeDtypeStruct((_N, _K), jnp.float32)
    w = _full_spec(_K, _K)
    b = _full_spec(1, _K)
    r = _row_spec(_BLK, _K)
    return pl.pallas_call(
        _layer1_body,
        grid=(grid,),
        in_specs=[r, r, w, w, w, b, r, r, w, w, w, b,
                  _user_spec(_BLK, _K), _item_spec(_BLK, _K)],
        out_specs=[r, r, r, r],
        out_shape=[out, out, out, out],
    )(sv, xpv, pwv, gwv, lwv, bv, st, xpt, pwt, gwt, lwt, bt, gu, gi)


def _final(a, b, c, d):
    grid = _N // _BLK
    r = _row_spec(_BLK, _K)
    return pl.pallas_call(
        _final_body,
        grid=(grid,),
        in_specs=[r, r, r, r],
        out_specs=r,
        out_shape=jax.ShapeDtypeStruct((_N, _K), jnp.float32),
    )(a, b, c, d)


def kernel(edge_index, Gu, Gi, feat_visual, Gum_visual, proj_W_visual, proj_b_visual, prop_W_visual_0, lin_W_visual_0, lin_b_visual_0, g_W_visual_0, g_b_visual_0, prop_W_visual_1, lin_W_visual_1, lin_b_visual_1, g_W_visual_1, g_b_visual_1, feat_textual, Gum_textual, proj_W_textual, proj_b_textual, prop_W_textual_0, lin_W_textual_0, lin_b_textual_0, g_W_textual_0, g_b_textual_0, prop_W_textual_1, lin_W_textual_1, lin_b_textual_1, g_W_textual_1, g_b_textual_1):
    npad = _EPAD - _E
    pad_src = jnp.arange(npad, dtype=jnp.int32) & 16383
    pad_dst = jnp.full((npad,), _N, jnp.int32) + (jnp.arange(npad, dtype=jnp.int32) & 15)
    src2d = jnp.concatenate([edge_index[0], pad_src]).reshape(_EPAD // _EROW, _EROW)
    dst2d = jnp.concatenate([edge_index[1], pad_dst]).reshape(_EPAD // _EROW, _EROW)
    zz = jnp.zeros((_ACC, _K), jnp.float32)

    csrc, cdst, counts = _edge_partition_sc(src2d, dst2d)
    csrc = csrc.reshape(2, _NSTRIP, _CROWS, _EROW)
    cdst = cdst.reshape(2, _NSTRIP, _CROWS, _EROW)

    bias0_v = (lin_b_visual_0 + g_b_visual_0).reshape(1, _K)
    bias1_v = (lin_b_visual_1 + g_b_visual_1).reshape(1, _K)
    bias0_t = (lin_b_textual_0 + g_b_textual_0).reshape(1, _K)
    bias1_t = (lin_b_textual_1 + g_b_textual_1).reshape(1, _K)

    z0_v, xh0_v, z0_t, xh0_t = _stage0(
        Gum_visual, feat_visual, proj_W_visual, proj_b_visual.reshape(1, _K),
        prop_W_visual_0, g_W_visual_0, lin_W_visual_0, bias0_v,
        Gum_textual, feat_textual, proj_W_textual, proj_b_textual.reshape(1, _K),
        prop_W_textual_0, g_W_textual_0, lin_W_textual_0, bias0_t, Gu, Gi)

    s0_v = _segment_sum_compact_sc(z0_v, csrc, cdst, counts, zz)
    s0_t = _segment_sum_compact_sc(z0_t, csrc, cdst, counts, zz)

    z1_v, xh1_v, z1_t, xh1_t = _layer1(
        s0_v, xh0_v, prop_W_visual_1, g_W_visual_1, lin_W_visual_1, bias1_v,
        s0_t, xh0_t, prop_W_textual_1, g_W_textual_1, lin_W_textual_1, bias1_t,
        Gu, Gi)

    s1_v = _segment_sum_compact_sc(z1_v, csrc, cdst, counts, zz)
    s1_t = _segment_sum_compact_sc(z1_t, csrc, cdst, counts, zz)

    x_all = _final(s1_v, xh1_v, s1_t, xh1_t)
    return x_all[:_NU], x_all[_NU:]


# speedup vs baseline: 1.8352x; 1.0399x over previous
import functools

import jax
import jax.numpy as jnp
from jax import lax
from jax.experimental import pallas as pl
from jax.experimental.pallas import tpu as pltpu
from jax.experimental.pallas import tpu_sc as plsc

_NU = 25000
_NI = 25000
_N = _NU + _NI
_E = 800000
_K = 64
_BLK = 1000

_HALF = 25000
_ACC = 25088
_ROWS_PER_TILE = _ACC // 16
_EROW = 128
_GRP = 2
_EPAD = 819200
_NGRP = _EPAD // (_EROW * _GRP)
_WB = 1000


_RPT = _EPAD // _EROW // 16
_IB = 8
_NB = _RPT // _IB

_NSTRIP = 32
_SROWS = _EPAD // _EROW // _NSTRIP
_FIB = 10
_PADU = 2048
_CAP = 28672
_CROWS = _CAP // _EROW


def _seg_body(z, srcm, dstm, zz, out, sidx, dloc, rows, acc,
              isem0, isem1, gsem0, gsem1, ssem0, ssem1):
    c = lax.axis_index("c")
    s = lax.axis_index("s")
    base = c * _HALF
    isem = (isem0, isem1)
    gsem = (gsem0, gsem1)
    ssem = (ssem0, ssem1)
    tile_r0 = s * _RPT

    pltpu.sync_copy(zz.at[pl.ds(s * _ROWS_PER_TILE, _ROWS_PER_TILE)],
                    acc.at[pl.ds(s * _ROWS_PER_TILE, _ROWS_PER_TILE)])
    plsc.subcore_barrier()

    iota = lax.iota(jnp.int32, 16)

    def remap(p):
        for r in range(_IB):
            for i in range(_EROW // 16):
                d = dloc[p, r, pl.ds(i * 16, 16)]
                loc = d - base
                m = (loc >= 0) & (loc < _HALF)
                dloc[p, r, pl.ds(i * 16, 16)] = jnp.where(m, loc, _HALF + iota)

    def issue_idx(p, blk):
        r0 = tile_r0 + blk * _IB
        pltpu.async_copy(srcm.at[pl.ds(r0, _IB)], sidx.at[p], isem[p])
        pltpu.async_copy(dstm.at[pl.ds(r0, _IB)], dloc.at[p], isem[p])

    def wait_idx(p):
        pltpu.make_async_copy(srcm.at[pl.ds(0, _IB)], sidx.at[p], isem[p]).wait()
        pltpu.make_async_copy(dstm.at[pl.ds(0, _IB)], dloc.at[p], isem[p]).wait()

    def issue_gather(p, r, q):
        pltpu.async_copy(z.at[sidx.at[p, r]], rows.at[q], gsem[q])

    def wait_gather(q):
        pltpu.make_async_copy(z.at[sidx.at[0, 0]], rows.at[q], gsem[q]).wait()

    pltpu.sync_copy(srcm.at[pl.ds(tile_r0, _IB)], sidx.at[0])
    pltpu.sync_copy(dstm.at[pl.ds(tile_r0, _IB)], dloc.at[0])
    remap(0)
    issue_idx(1, 1)
    issue_gather(0, 0, 0)

    def loop(i, carry):
        for p in (0, 1):
            x = 2 * i + p
            pn = p ^ 1

            @pl.when(x + 1 < _NB)
            def _():
                wait_idx(pn)
                remap(pn)
            for r in range(_IB):
                q = r & 1
                g = x * _IB + r

                @pl.when(g + 1 < _RPT)
                def _():
                    if r < _IB - 1:
                        issue_gather(p, r + 1, q ^ 1)
                    else:
                        issue_gather(pn, 0, q ^ 1)
                wait_gather(q)
                pltpu.sync_copy(rows.at[q], acc.at[dloc.at[p, r]], add=True)

            @pl.when(x + 2 < _NB)
            def _():
                issue_idx(p, x + 2)
        return carry

    lax.fori_loop(0, _NB // 2, loop, 0)
    plsc.subcore_barrier()

    for k2 in range(( _HALF // _WB + 15) // 16):
        g = s + 16 * k2

        @pl.when(g < _HALF // _WB)
        def _():
            pltpu.sync_copy(acc.at[pl.ds(g * _WB, _WB)],
                            out.at[pl.ds(base + g * _WB, _WB)])


def _filter_body(srcm, dstm, csrc, cdst, counts, isrc, idst,
                 bsrc0, bdst0, bsrc1, bdst1, cb, isem0, isem1):
    c = lax.axis_index("c")
    s = lax.axis_index("s")
    w = c * 16 + s
    strip0 = w * _SROWS
    isem = (isem0, isem1)
    iota = lax.iota(jnp.int32, 16)
    bufs = ((bsrc0, bdst0), (bsrc1, bdst1))

    def issue_idx(p, blk):
        r0 = strip0 + blk * _FIB
        pltpu.async_copy(srcm.at[pl.ds(r0, _FIB)], isrc.at[p], isem[p])
        pltpu.async_copy(dstm.at[pl.ds(r0, _FIB)], idst.at[p], isem[p])

    def wait_idx(p):
        pltpu.make_async_copy(srcm.at[pl.ds(0, _FIB)], isrc.at[p], isem[p]).wait()
        pltpu.make_async_copy(dstm.at[pl.ds(0, _FIB)], idst.at[p], isem[p]).wait()

    pltpu.sync_copy(srcm.at[pl.ds(strip0, _FIB)], isrc.at[0])
    pltpu.sync_copy(dstm.at[pl.ds(strip0, _FIB)], idst.at[0])
    issue_idx(1, 1)

    nblk = _SROWS // _FIB

    def loop(i, carry):
        n0, n1 = carry
        for p in (0, 1):
            x = 2 * i + p
            pn = p ^ 1

            @pl.when(x + 1 < nblk)
            def _():
                wait_idx(pn)
            for r in range(_FIB):
                for v in range(_EROW // 16):
                    sv = isrc[p, r, pl.ds(v * 16, 16)]
                    dv = idst[p, r, pl.ds(v * 16, 16)]
                    m0 = dv < _HALF
                    pref0 = plsc.cumsum(m0.astype(jnp.int32))
                    pos0 = n0 + pref0 - 1
                    plsc.store_scatter(bsrc0, [pos0], sv, mask=m0)
                    plsc.store_scatter(bdst0, [pos0], dv, mask=m0)
                    n0 = n0 + jnp.max(pref0)
                    loc = dv - _HALF
                    m1 = (loc >= 0) & (loc < _HALF)
                    pref1 = plsc.cumsum(m1.astype(jnp.int32))
                    pos1 = n1 + pref1 - 1
                    plsc.store_scatter(bsrc1, [pos1], sv, mask=m1)
                    plsc.store_scatter(bdst1, [pos1], loc, mask=m1)
                    n1 = n1 + jnp.max(pref1)

            @pl.when(x + 2 < nblk)
            def _():
                issue_idx(p, x + 2)
        return (n0, n1)

    n0, n1 = lax.fori_loop(0, nblk // 2, loop,
                           (jnp.int32(0), jnp.int32(0)))

    for h, n in ((0, n0), (1, n1)):
        bs, bd = bufs[h]
        for i in range(_PADU // 16):
            src_pad = (iota + 16 * i) & 16383
            bs[pl.ds(n + 16 * i, 16)] = src_pad
            bd[pl.ds(n + 16 * i, 16)] = _HALF + iota
        npairs = (n + _PADU - 1) >> 11
        cb[...] = jnp.broadcast_to(npairs, (16,)).astype(jnp.int32)
        pltpu.sync_copy(cb, counts.at[h, w])
        pltpu.sync_copy(bs, csrc.at[h, w])
        pltpu.sync_copy(bd, cdst.at[h, w])


@jax.jit
def _edge_partition_sc(src2d, dst2d):
    mesh = plsc.VectorSubcoreMesh(core_axis_name="c", subcore_axis_name="s")
    return pl.kernel(
        _filter_body,
        out_type=(
            jax.ShapeDtypeStruct((2, _NSTRIP, _CAP), jnp.int32),
            jax.ShapeDtypeStruct((2, _NSTRIP, _CAP), jnp.int32),
            jax.ShapeDtypeStruct((2, _NSTRIP, 16), jnp.int32),
        ),
        mesh=mesh,
        scratch_types=[
            pltpu.VMEM((2, _FIB, _EROW), jnp.int32),
            pltpu.VMEM((2, _FIB, _EROW), jnp.int32),
            pltpu.VMEM((_CAP,), jnp.int32),
            pltpu.VMEM((_CAP,), jnp.int32),
            pltpu.VMEM((_CAP,), jnp.int32),
            pltpu.VMEM((_CAP,), jnp.int32),
            pltpu.VMEM((16,), jnp.int32),
            pltpu.SemaphoreType.DMA,
            pltpu.SemaphoreType.DMA,
        ],
        compiler_params=pltpu.CompilerParams(use_tc_tiling_on_sc=False,
                                             needs_layout_passes=False),
    )(src2d, dst2d)


def _seg_compact_body(z, csrc, cdst, counts, zz, out, sidx, dloc, rows, cntv,
                      acc, isem0, isem1, gsem0, gsem1):
    c = lax.axis_index("c")
    s = lax.axis_index("s")
    base = c * _HALF
    isem = (isem0, isem1)
    gsem = (gsem0, gsem1)

    pltpu.sync_copy(zz.at[pl.ds(s * _ROWS_PER_TILE, _ROWS_PER_TILE)],
                    acc.at[pl.ds(s * _ROWS_PER_TILE, _ROWS_PER_TILE)])
    plsc.subcore_barrier()

    def issue_gather(p, r, q):
        pltpu.async_copy(z.at[sidx.at[p, r]], rows.at[q], gsem[q])

    def wait_gather(q):
        pltpu.make_async_copy(z.at[sidx.at[0, 0]], rows.at[q], gsem[q]).wait()

    for reg in (0, 1):
        w = 2 * s + reg
        pltpu.sync_copy(counts.at[c, w], cntv)
        npairs = jnp.max(cntv[...])
        nblocks = npairs * 2
        nrows = nblocks * _IB

        def issue_idx(p, blk):
            pltpu.async_copy(csrc.at[c, w, pl.ds(blk * _IB, _IB)],
                             sidx.at[p], isem[p])
            pltpu.async_copy(cdst.at[c, w, pl.ds(blk * _IB, _IB)],
                             dloc.at[p], isem[p])

        def wait_idx(p):
            pltpu.make_async_copy(csrc.at[c, w, pl.ds(0, _IB)],
                                  sidx.at[p], isem[p]).wait()
            pltpu.make_async_copy(cdst.at[c, w, pl.ds(0, _IB)],
                                  dloc.at[p], isem[p]).wait()

        pltpu.sync_copy(csrc.at[c, w, pl.ds(0, _IB)], sidx.at[0])
        pltpu.sync_copy(cdst.at[c, w, pl.ds(0, _IB)], dloc.at[0])

        @pl.when(nblocks > 1)
        def _():
            issue_idx(1, 1)
        issue_gather(0, 0, 0)

        def loop(bp, carry):
            for p in (0, 1):
                x = 2 * bp + p
                pn = p ^ 1

                @pl.when(x + 1 < nblocks)
                def _():
                    wait_idx(pn)
                for r in range(_IB):
                    q = r & 1
                    g = x * _IB + r

                    @pl.when(g + 1 < nrows)
                    def _():
                        if r < _IB - 1:
                            issue_gather(p, r + 1, q ^ 1)
                        else:
                            issue_gather(pn, 0, q ^ 1)
                    wait_gather(q)
                    pltpu.sync_copy(rows.at[q], acc.at[dloc.at[p, r]],
                                    add=True)

                @pl.when(x + 2 < nblocks)
                def _():
                    issue_idx(p, x + 2)
            return carry

        lax.fori_loop(0, npairs, loop, 0)

    plsc.subcore_barrier()

    for k2 in range(( _HALF // _WB + 15) // 16):
        g = s + 16 * k2

        @pl.when(g < _HALF // _WB)
        def _():
            pltpu.sync_copy(acc.at[pl.ds(g * _WB, _WB)],
                            out.at[pl.ds(base + g * _WB, _WB)])


@jax.jit
def _segment_sum_compact_sc(z, csrc, cdst, counts, zz):
    mesh = plsc.VectorSubcoreMesh(core_axis_name="c", subcore_axis_name="s")
    return pl.kernel(
        _seg_compact_body,
        out_type=jax.ShapeDtypeStruct((_N, _K), jnp.float32),
        mesh=mesh,
        scratch_types=[
            pltpu.VMEM((2, _IB, _EROW), jnp.int32),
            pltpu.VMEM((2, _IB, _EROW), jnp.int32),
            pltpu.VMEM((2, _EROW, _K), jnp.float32),
            pltpu.VMEM((16,), jnp.int32),
            pltpu.VMEM_SHARED((_ACC, _K), jnp.float32),
            pltpu.SemaphoreType.DMA,
            pltpu.SemaphoreType.DMA,
            pltpu.SemaphoreType.DMA,
            pltpu.SemaphoreType.DMA,
        ],
        compiler_params=pltpu.CompilerParams(use_tc_tiling_on_sc=False,
                                             needs_layout_passes=False),
    )(z, csrc, cdst, counts, zz)


@jax.jit
def _segment_sum_sc(z, src2d, dst2d, zz):
    mesh = plsc.VectorSubcoreMesh(core_axis_name="c", subcore_axis_name="s")
    return pl.kernel(
        _seg_body,
        out_type=jax.ShapeDtypeStruct((_N, _K), jnp.float32),
        mesh=mesh,
        scratch_types=[
            pltpu.VMEM((2, _IB, _EROW), jnp.int32),
            pltpu.VMEM((2, _IB, _EROW), jnp.int32),
            pltpu.VMEM((2, _EROW, _K), jnp.float32),
            pltpu.VMEM_SHARED((_ACC, _K), jnp.float32),
            pltpu.SemaphoreType.DMA,
            pltpu.SemaphoreType.DMA,
            pltpu.SemaphoreType.DMA,
            pltpu.SemaphoreType.DMA,
            pltpu.SemaphoreType.DMA,
            pltpu.SemaphoreType.DMA,
        ],
        compiler_params=pltpu.CompilerParams(use_tc_tiling_on_sc=False),
    )(z, src2d, dst2d, zz)


def _dotT(a, b):
    return lax.dot_general(a, b, (((1,), (1,)), ((), ())),
                           preferred_element_type=jnp.float32)


def _normalize(x):
    nrm = jnp.sqrt(jnp.sum(x * x, axis=1, keepdims=True))
    return x / jnp.maximum(nrm, 1e-12)


def _layer_core(x, pw, gw, lw, bias, ego):
    weff = _dotT(pw, gw)
    z = jnp.dot(x, weff, preferred_element_type=jnp.float32)
    xh = _dotT(x, lw) + bias + ego
    return z, xh


def _stage0_body(gumv, fv, pjwv, pjbv, pwv, gwv, lwv, bv,
                 gumt, ft, pjwt, pjbt, pwt, gwt, lwt, bt, gu, gi,
                 zv, xhv, zt, xht):
    user = pl.program_id(0) < _NU // _BLK
    e = jnp.where(user, gu[...], gi[...])
    for (gum, f, pjw, pjb, pw, gw, lw, b, z, xh) in (
            (gumv, fv, pjwv, pjbv, pwv, gwv, lwv, bv, zv, xhv),
            (gumt, ft, pjwt, pjbt, pwt, gwt, lwt, bt, zt, xht)):
        xg = _normalize(gum[...])
        xp = _normalize(_dotT(f[...], pjw[...]) + pjb[...])
        x = jnp.where(user, xg, xp)
        z[...], xh[...] = _layer_core(x, pw[...], gw[...], lw[...], b[...], e)


def _layer1_body(s, xp, pw, gw, lw, b, gu, gi, z, xh):
    user = pl.program_id(0) < _NU // _BLK
    e = jnp.where(user, gu[...], gi[...])
    z[...], xh[...] = _layer_core(s[...] + xp[...], pw[...], gw[...],
                                  lw[...], b[...], e)


def _add3_body(a, b, c, o):
    o[...] = a[...] + b[...] + c[...]


def _add2_body(a, b, o):
    o[...] = a[...] + b[...]


def _row_spec(blk, k):
    return pl.BlockSpec((blk, k), lambda i: (i, 0))


def _full_spec(r, k):
    return pl.BlockSpec((r, k), lambda i: (0, 0))


def _user_spec(blk, k):
    return pl.BlockSpec((blk, k), lambda i: (jnp.minimum(i, _NU // _BLK - 1), 0))


def _item_spec(blk, k):
    return pl.BlockSpec((blk, k), lambda i: (jnp.maximum(i - _NU // _BLK, 0), 0))


def _stage0(gumv, fv, pjwv, pjbv, pwv, gwv, lwv, bv,
            gumt, ft, pjwt, pjbt, pwt, gwt, lwt, bt, gu, gi):
    grid = _N // _BLK
    out = jax.ShapeDtypeStruct((_N, _K), jnp.float32)
    w = _full_spec(_K, _K)
    b = _full_spec(1, _K)
    r = _row_spec(_BLK, _K)
    mod = [_user_spec(_BLK, _K), _item_spec(_BLK, 128), _full_spec(_K, 128),
           b, w, w, w, b]
    return pl.pallas_call(
        _stage0_body,
        grid=(grid,),
        in_specs=mod + mod + [_user_spec(_BLK, _K), _item_spec(_BLK, _K)],
        out_specs=[r, r, r, r],
        out_shape=[out, out, out, out],
    )(gumv, fv, pjwv, pjbv, pwv, gwv, lwv, bv,
      gumt, ft, pjwt, pjbt, pwt, gwt, lwt, bt, gu, gi)


def _layer1(s, xp, pw, gw, lw, b, gu, gi):
    grid = _N // _BLK
    out = jax.ShapeDtypeStruct((_N, _K), jnp.float32)
    w = _full_spec(_K, _K)
    bb = _full_spec(1, _K)
    r = _row_spec(_BLK, _K)
    return pl.pallas_call(
        _layer1_body,
        grid=(grid,),
        in_specs=[r, r, w, w, w, bb,
                  _user_spec(_BLK, _K), _item_spec(_BLK, _K)],
        out_specs=[r, r],
        out_shape=[out, out],
    )(s, xp, pw, gw, lw, b, gu, gi)


def _add3(a, b, c):
    grid = _N // _BLK
    r = _row_spec(_BLK, _K)
    return pl.pallas_call(
        _add3_body,
        grid=(grid,),
        in_specs=[r, r, r],
        out_specs=r,
        out_shape=jax.ShapeDtypeStruct((_N, _K), jnp.float32),
    )(a, b, c)


def _add2(a, b):
    grid = _N // _BLK
    r = _row_spec(_BLK, _K)
    return pl.pallas_call(
        _add2_body,
        grid=(grid,),
        in_specs=[r, r],
        out_specs=r,
        out_shape=jax.ShapeDtypeStruct((_N, _K), jnp.float32),
    )(a, b)


def kernel(edge_index, Gu, Gi, feat_visual, Gum_visual, proj_W_visual, proj_b_visual, prop_W_visual_0, lin_W_visual_0, lin_b_visual_0, g_W_visual_0, g_b_visual_0, prop_W_visual_1, lin_W_visual_1, lin_b_visual_1, g_W_visual_1, g_b_visual_1, feat_textual, Gum_textual, proj_W_textual, proj_b_textual, prop_W_textual_0, lin_W_textual_0, lin_b_textual_0, g_W_textual_0, g_b_textual_0, prop_W_textual_1, lin_W_textual_1, lin_b_textual_1, g_W_textual_1, g_b_textual_1):
    npad = _EPAD - _E
    pad_src = jnp.arange(npad, dtype=jnp.int32) & 16383
    pad_dst = jnp.full((npad,), _N, jnp.int32) + (jnp.arange(npad, dtype=jnp.int32) & 15)
    src2d = jnp.concatenate([edge_index[0], pad_src]).reshape(_EPAD // _EROW, _EROW)
    dst2d = jnp.concatenate([edge_index[1], pad_dst]).reshape(_EPAD // _EROW, _EROW)
    zz = jnp.zeros((_ACC, _K), jnp.float32)

    csrc, cdst, counts = _edge_partition_sc(src2d, dst2d)
    csrc = csrc.reshape(2, _NSTRIP, _CROWS, _EROW)
    cdst = cdst.reshape(2, _NSTRIP, _CROWS, _EROW)

    bias0_v = (lin_b_visual_0 + g_b_visual_0).reshape(1, _K)
    bias1_v = (lin_b_visual_1 + g_b_visual_1).reshape(1, _K)
    bias0_t = (lin_b_textual_0 + g_b_textual_0).reshape(1, _K)
    bias1_t = (lin_b_textual_1 + g_b_textual_1).reshape(1, _K)

    z0_v, xh0_v, z0_t, xh0_t = _stage0(
        Gum_visual, feat_visual, proj_W_visual, proj_b_visual.reshape(1, _K),
        prop_W_visual_0, g_W_visual_0, lin_W_visual_0, bias0_v,
        Gum_textual, feat_textual, proj_W_textual, proj_b_textual.reshape(1, _K),
        prop_W_textual_0, g_W_textual_0, lin_W_textual_0, bias0_t, Gu, Gi)

    s0_v = _segment_sum_compact_sc(z0_v, csrc, cdst, counts, zz)
    s0_t = _segment_sum_compact_sc(z0_t, csrc, cdst, counts, zz)
    z1_v, xh1_v = _layer1(s0_v, xh0_v, prop_W_visual_1, g_W_visual_1,
                          lin_W_visual_1, bias1_v, Gu, Gi)

    s1_v = _segment_sum_compact_sc(z1_v, csrc, cdst, counts, zz)
    z1_t, xh1_t = _layer1(s0_t, xh0_t, prop_W_textual_1, g_W_textual_1,
                          lin_W_textual_1, bias1_t, Gu, Gi)

    s1_t = _segment_sum_compact_sc(z1_t, csrc, cdst, counts, zz)
    part = _add3(s1_v, xh1_v, xh1_t)

    x_all = _add2(part, s1_t)
    return x_all[:_NU], x_all[_NU:]


# speedup vs baseline: 2.0145x; 1.0977x over previous
import functools

import jax
import jax.numpy as jnp
from jax import lax
from jax.experimental import pallas as pl
from jax.experimental.pallas import tpu as pltpu
from jax.experimental.pallas import tpu_sc as plsc

_NU = 25000
_NI = 25000
_N = _NU + _NI
_E = 800000
_K = 64
_BLK = 1000

_HALF = 25000
_ACC = 25024
_ROWS_PER_TILE = _ACC // 16
_EROW = 128
_GRP = 2
_EPAD = 819200
_NGRP = _EPAD // (_EROW * _GRP)
_WB = 1000


_RPT = _EPAD // _EROW // 16
_IB = 8
_NB = _RPT // _IB

_NSTRIP = 32
_SROWS = _EPAD // _EROW // _NSTRIP
_FIB = 10
_PADU = 3072
_CAP = 28672
_CROWS = _CAP // _EROW


def _seg_body(z, srcm, dstm, zz, out, sidx, dloc, rows, acc,
              isem0, isem1, gsem0, gsem1, ssem0, ssem1):
    c = lax.axis_index("c")
    s = lax.axis_index("s")
    base = c * _HALF
    isem = (isem0, isem1)
    gsem = (gsem0, gsem1)
    ssem = (ssem0, ssem1)
    tile_r0 = s * _RPT

    pltpu.sync_copy(zz.at[pl.ds(s * _ROWS_PER_TILE, _ROWS_PER_TILE)],
                    acc.at[pl.ds(s * _ROWS_PER_TILE, _ROWS_PER_TILE)])
    plsc.subcore_barrier()

    iota = lax.iota(jnp.int32, 16)

    def remap(p):
        for r in range(_IB):
            for i in range(_EROW // 16):
                d = dloc[p, r, pl.ds(i * 16, 16)]
                loc = d - base
                m = (loc >= 0) & (loc < _HALF)
                dloc[p, r, pl.ds(i * 16, 16)] = jnp.where(m, loc, _HALF + iota)

    def issue_idx(p, blk):
        r0 = tile_r0 + blk * _IB
        pltpu.async_copy(srcm.at[pl.ds(r0, _IB)], sidx.at[p], isem[p])
        pltpu.async_copy(dstm.at[pl.ds(r0, _IB)], dloc.at[p], isem[p])

    def wait_idx(p):
        pltpu.make_async_copy(srcm.at[pl.ds(0, _IB)], sidx.at[p], isem[p]).wait()
        pltpu.make_async_copy(dstm.at[pl.ds(0, _IB)], dloc.at[p], isem[p]).wait()

    def issue_gather(p, r, q):
        pltpu.async_copy(z.at[sidx.at[p, r]], rows.at[q], gsem[q])

    def wait_gather(q):
        pltpu.make_async_copy(z.at[sidx.at[0, 0]], rows.at[q], gsem[q]).wait()

    pltpu.sync_copy(srcm.at[pl.ds(tile_r0, _IB)], sidx.at[0])
    pltpu.sync_copy(dstm.at[pl.ds(tile_r0, _IB)], dloc.at[0])
    remap(0)
    issue_idx(1, 1)
    issue_gather(0, 0, 0)

    def loop(i, carry):
        for p in (0, 1):
            x = 2 * i + p
            pn = p ^ 1

            @pl.when(x + 1 < _NB)
            def _():
                wait_idx(pn)
                remap(pn)
            for r in range(_IB):
                q = r & 1
                g = x * _IB + r

                @pl.when(g + 1 < _RPT)
                def _():
                    if r < _IB - 1:
                        issue_gather(p, r + 1, q ^ 1)
                    else:
                        issue_gather(pn, 0, q ^ 1)
                wait_gather(q)
                pltpu.sync_copy(rows.at[q], acc.at[dloc.at[p, r]], add=True)

            @pl.when(x + 2 < _NB)
            def _():
                issue_idx(p, x + 2)
        return carry

    lax.fori_loop(0, _NB // 2, loop, 0)
    plsc.subcore_barrier()

    for k2 in range(( _HALF // _WB + 15) // 16):
        g = s + 16 * k2

        @pl.when(g < _HALF // _WB)
        def _():
            pltpu.sync_copy(acc.at[pl.ds(g * _WB, _WB)],
                            out.at[pl.ds(base + g * _WB, _WB)])


def _filter_body(srcm, dstm, csrc, cdst, counts, isrc, idst,
                 bsrc0, bdst0, bsrc1, bdst1, cb, isem0, isem1):
    c = lax.axis_index("c")
    s = lax.axis_index("s")
    w = c * 16 + s
    strip0 = w * _SROWS
    isem = (isem0, isem1)
    iota = lax.iota(jnp.int32, 16)
    bufs = ((bsrc0, bdst0), (bsrc1, bdst1))

    def issue_idx(p, blk):
        r0 = strip0 + blk * _FIB
        pltpu.async_copy(srcm.at[pl.ds(r0, _FIB)], isrc.at[p], isem[p])
        pltpu.async_copy(dstm.at[pl.ds(r0, _FIB)], idst.at[p], isem[p])

    def wait_idx(p):
        pltpu.make_async_copy(srcm.at[pl.ds(0, _FIB)], isrc.at[p], isem[p]).wait()
        pltpu.make_async_copy(dstm.at[pl.ds(0, _FIB)], idst.at[p], isem[p]).wait()

    pltpu.sync_copy(srcm.at[pl.ds(strip0, _FIB)], isrc.at[0])
    pltpu.sync_copy(dstm.at[pl.ds(strip0, _FIB)], idst.at[0])
    issue_idx(1, 1)

    nblk = _SROWS // _FIB

    def loop(i, carry):
        n0, n1 = carry
        for p in (0, 1):
            x = 2 * i + p
            pn = p ^ 1

            @pl.when(x + 1 < nblk)
            def _():
                wait_idx(pn)
            for r in range(_FIB):
                for v in range(_EROW // 16):
                    sv = isrc[p, r, pl.ds(v * 16, 16)]
                    dv = idst[p, r, pl.ds(v * 16, 16)]
                    m0 = dv < _HALF
                    pref0 = plsc.cumsum(m0.astype(jnp.int32))
                    pos0 = n0 + pref0 - 1
                    plsc.store_scatter(bsrc0, [pos0], sv, mask=m0)
                    plsc.store_scatter(bdst0, [pos0], dv, mask=m0)
                    n0 = n0 + jnp.max(pref0)
                    loc = dv - _HALF
                    m1 = (loc >= 0) & (loc < _HALF)
                    pref1 = plsc.cumsum(m1.astype(jnp.int32))
                    pos1 = n1 + pref1 - 1
                    plsc.store_scatter(bsrc1, [pos1], sv, mask=m1)
                    plsc.store_scatter(bdst1, [pos1], loc, mask=m1)
                    n1 = n1 + jnp.max(pref1)

            @pl.when(x + 2 < nblk)
            def _():
                issue_idx(p, x + 2)
        return (n0, n1)

    n0, n1 = lax.fori_loop(0, nblk // 2, loop,
                           (jnp.int32(0), jnp.int32(0)))

    for h, n in ((0, n0), (1, n1)):
        bs, bd = bufs[h]
        for i in range(_PADU // 16):
            src_pad = (iota + 16 * i) & 16383
            bs[pl.ds(n + 16 * i, 16)] = src_pad
            bd[pl.ds(n + 16 * i, 16)] = _HALF + iota
        thr = iota * _PADU
        nvec = jnp.broadcast_to(n, (16,)).astype(jnp.int32)
        ntr = jnp.maximum(
            jnp.max(plsc.all_reduce_population_count(nvec > thr)), 1)
        cb[...] = jnp.broadcast_to(ntr, (16,)).astype(jnp.int32)
        pltpu.sync_copy(cb, counts.at[h, w])
        pltpu.sync_copy(bs, csrc.at[h, w])
        pltpu.sync_copy(bd, cdst.at[h, w])


@jax.jit
def _edge_partition_sc(src2d, dst2d):
    mesh = plsc.VectorSubcoreMesh(core_axis_name="c", subcore_axis_name="s")
    return pl.kernel(
        _filter_body,
        out_type=(
            jax.ShapeDtypeStruct((2, _NSTRIP, _CAP), jnp.int32),
            jax.ShapeDtypeStruct((2, _NSTRIP, _CAP), jnp.int32),
            jax.ShapeDtypeStruct((2, _NSTRIP, 16), jnp.int32),
        ),
        mesh=mesh,
        scratch_types=[
            pltpu.VMEM((2, _FIB, _EROW), jnp.int32),
            pltpu.VMEM((2, _FIB, _EROW), jnp.int32),
            pltpu.VMEM((_CAP,), jnp.int32),
            pltpu.VMEM((_CAP,), jnp.int32),
            pltpu.VMEM((_CAP,), jnp.int32),
            pltpu.VMEM((_CAP,), jnp.int32),
            pltpu.VMEM((16,), jnp.int32),
            pltpu.SemaphoreType.DMA,
            pltpu.SemaphoreType.DMA,
        ],
        compiler_params=pltpu.CompilerParams(use_tc_tiling_on_sc=False,
                                             needs_layout_passes=False),
    )(src2d, dst2d)


def _seg_compact_body(z, csrc, cdst, counts, zz, out, sidx, dloc, rows, cntv,
                      acc, isem0, isem1, isem2, gsem0, gsem1, gsem2):
    c = lax.axis_index("c")
    s = lax.axis_index("s")
    base = c * _HALF
    isem = (isem0, isem1, isem2)
    gsem = (gsem0, gsem1, gsem2)

    pltpu.sync_copy(zz.at[pl.ds(s * _ROWS_PER_TILE, _ROWS_PER_TILE)],
                    acc.at[pl.ds(s * _ROWS_PER_TILE, _ROWS_PER_TILE)])
    plsc.subcore_barrier()

    def issue_gather(p, r, q):
        pltpu.async_copy(z.at[sidx.at[p, r]], rows.at[q], gsem[q])

    def wait_gather(q):
        pltpu.make_async_copy(z.at[sidx.at[0, 0]], rows.at[q], gsem[q]).wait()

    for reg in (0, 1):
        w = 2 * s + reg
        pltpu.sync_copy(counts.at[c, w], cntv)
        ntr = jnp.max(cntv[...])
        nblocks = ntr * 3
        nrows = nblocks * _IB

        def issue_idx(p, blk):
            pltpu.async_copy(csrc.at[c, w, pl.ds(blk * _IB, _IB)],
                             sidx.at[p], isem[p])
            pltpu.async_copy(cdst.at[c, w, pl.ds(blk * _IB, _IB)],
                             dloc.at[p], isem[p])

        def wait_idx(p):
            pltpu.make_async_copy(csrc.at[c, w, pl.ds(0, _IB)],
                                  sidx.at[p], isem[p]).wait()
            pltpu.make_async_copy(cdst.at[c, w, pl.ds(0, _IB)],
                                  dloc.at[p], isem[p]).wait()

        pltpu.sync_copy(csrc.at[c, w, pl.ds(0, _IB)], sidx.at[0])
        pltpu.sync_copy(cdst.at[c, w, pl.ds(0, _IB)], dloc.at[0])
        issue_idx(1, 1)
        issue_idx(2, 2)
        issue_gather(0, 0, 0)
        issue_gather(0, 1, 1)

        def loop(t, carry):
            for p in (0, 1, 2):
                x = 3 * t + p

                @pl.when(x + 1 < nblocks)
                def _():
                    wait_idx((p + 1) % 3)
                for r in range(_IB):
                    g = x * _IB + r
                    q = (p * 2 + r) % 3
                    wait_gather(q)

                    @pl.when(g + 2 < nrows)
                    def _():
                        if r < _IB - 2:
                            issue_gather(p, r + 2, (q + 2) % 3)
                        else:
                            issue_gather((p + 1) % 3, r - (_IB - 2),
                                         (q + 2) % 3)
                    pltpu.sync_copy(rows.at[q], acc.at[dloc.at[p, r]],
                                    add=True)

                @pl.when(x + 3 < nblocks)
                def _():
                    issue_idx(p, x + 3)
            return carry

        lax.fori_loop(0, ntr, loop, 0)

    plsc.subcore_barrier()

    for k2 in range(( _HALF // _WB + 15) // 16):
        g = s + 16 * k2

        @pl.when(g < _HALF // _WB)
        def _():
            pltpu.sync_copy(acc.at[pl.ds(g * _WB, _WB)],
                            out.at[pl.ds(base + g * _WB, _WB)])


@jax.jit
def _segment_sum_compact_sc(z, csrc, cdst, counts, zz):
    mesh = plsc.VectorSubcoreMesh(core_axis_name="c", subcore_axis_name="s")
    return pl.kernel(
        _seg_compact_body,
        out_type=jax.ShapeDtypeStruct((_N, _K), jnp.float32),
        mesh=mesh,
        scratch_types=[
            pltpu.VMEM((3, _IB, _EROW), jnp.int32),
            pltpu.VMEM((3, _IB, _EROW), jnp.int32),
            pltpu.VMEM((3, _EROW, _K), jnp.float32),
            pltpu.VMEM((16,), jnp.int32),
            pltpu.VMEM_SHARED((_ACC, _K), jnp.float32),
            pltpu.SemaphoreType.DMA,
            pltpu.SemaphoreType.DMA,
            pltpu.SemaphoreType.DMA,
            pltpu.SemaphoreType.DMA,
            pltpu.SemaphoreType.DMA,
            pltpu.SemaphoreType.DMA,
        ],
        compiler_params=pltpu.CompilerParams(use_tc_tiling_on_sc=False,
                                             needs_layout_passes=False),
    )(z, csrc, cdst, counts, zz)


@jax.jit
def _segment_sum_sc(z, src2d, dst2d, zz):
    mesh = plsc.VectorSubcoreMesh(core_axis_name="c", subcore_axis_name="s")
    return pl.kernel(
        _seg_body,
        out_type=jax.ShapeDtypeStruct((_N, _K), jnp.float32),
        mesh=mesh,
        scratch_types=[
            pltpu.VMEM((2, _IB, _EROW), jnp.int32),
            pltpu.VMEM((2, _IB, _EROW), jnp.int32),
            pltpu.VMEM((2, _EROW, _K), jnp.float32),
            pltpu.VMEM_SHARED((_ACC, _K), jnp.float32),
            pltpu.SemaphoreType.DMA,
            pltpu.SemaphoreType.DMA,
            pltpu.SemaphoreType.DMA,
            pltpu.SemaphoreType.DMA,
            pltpu.SemaphoreType.DMA,
            pltpu.SemaphoreType.DMA,
        ],
        compiler_params=pltpu.CompilerParams(use_tc_tiling_on_sc=False),
    )(z, src2d, dst2d, zz)


def _dotT(a, b):
    return lax.dot_general(a, b, (((1,), (1,)), ((), ())),
                           preferred_element_type=jnp.float32)


def _normalize(x):
    nrm = jnp.sqrt(jnp.sum(x * x, axis=1, keepdims=True))
    return x / jnp.maximum(nrm, 1e-12)


def _layer_core(x, pw, gw, lw, bias, ego):
    weff = _dotT(pw, gw)
    z = jnp.dot(x, weff, preferred_element_type=jnp.float32)
    xh = _dotT(x, lw) + bias + ego
    return z, xh


def _stage0_body(gumv, fv, pjwv, pjbv, pwv, gwv, lwv, bv,
                 gumt, ft, pjwt, pjbt, pwt, gwt, lwt, bt, gu, gi,
                 zv, xhv, zt, xht):
    user = pl.program_id(0) < _NU // _BLK
    e = jnp.where(user, gu[...], gi[...])
    for (gum, f, pjw, pjb, pw, gw, lw, b, z, xh) in (
            (gumv, fv, pjwv, pjbv, pwv, gwv, lwv, bv, zv, xhv),
            (gumt, ft, pjwt, pjbt, pwt, gwt, lwt, bt, zt, xht)):
        xg = _normalize(gum[...])
        xp = _normalize(_dotT(f[...], pjw[...]) + pjb[...])
        x = jnp.where(user, xg, xp)
        z[...], xh[...] = _layer_core(x, pw[...], gw[...], lw[...], b[...], e)


def _layer1_body(s, xp, pw, gw, lw, b, gu, gi, z, xh):
    user = pl.program_id(0) < _NU // _BLK
    e = jnp.where(user, gu[...], gi[...])
    z[...], xh[...] = _layer_core(s[...] + xp[...], pw[...], gw[...],
                                  lw[...], b[...], e)


def _add3_body(a, b, c, o):
    o[...] = a[...] + b[...] + c[...]


def _add2_body(a, b, o):
    o[...] = a[...] + b[...]


def _row_spec(blk, k):
    return pl.BlockSpec((blk, k), lambda i: (i, 0))


def _full_spec(r, k):
    return pl.BlockSpec((r, k), lambda i: (0, 0))


def _user_spec(blk, k):
    return pl.BlockSpec((blk, k), lambda i: (jnp.minimum(i, _NU // _BLK - 1), 0))


def _item_spec(blk, k):
    return pl.BlockSpec((blk, k), lambda i: (jnp.maximum(i - _NU // _BLK, 0), 0))


def _stage0(gumv, fv, pjwv, pjbv, pwv, gwv, lwv, bv,
            gumt, ft, pjwt, pjbt, pwt, gwt, lwt, bt, gu, gi):
    grid = _N // _BLK
    out = jax.ShapeDtypeStruct((_N, _K), jnp.float32)
    w = _full_spec(_K, _K)
    b = _full_spec(1, _K)
    r = _row_spec(_BLK, _K)
    mod = [_user_spec(_BLK, _K), _item_spec(_BLK, 128), _full_spec(_K, 128),
           b, w, w, w, b]
    return pl.pallas_call(
        _stage0_body,
        grid=(grid,),
        in_specs=mod + mod + [_user_spec(_BLK, _K), _item_spec(_BLK, _K)],
        out_specs=[r, r, r, r],
        out_shape=[out, out, out, out],
    )(gumv, fv, pjwv, pjbv, pwv, gwv, lwv, bv,
      gumt, ft, pjwt, pjbt, pwt, gwt, lwt, bt, gu, gi)


def _layer1(s, xp, pw, gw, lw, b, gu, gi):
    grid = _N // _BLK
    out = jax.ShapeDtypeStruct((_N, _K), jnp.float32)
    w = _full_spec(_K, _K)
    bb = _full_spec(1, _K)
    r = _row_spec(_BLK, _K)
    return pl.pallas_call(
        _layer1_body,
        grid=(grid,),
        in_specs=[r, r, w, w, w, bb,
                  _user_spec(_BLK, _K), _item_spec(_BLK, _K)],
        out_specs=[r, r],
        out_shape=[out, out],
    )(s, xp, pw, gw, lw, b, gu, gi)


def _add3(a, b, c):
    grid = _N // _BLK
    r = _row_spec(_BLK, _K)
    return pl.pallas_call(
        _add3_body,
        grid=(grid,),
        in_specs=[r, r, r],
        out_specs=r,
        out_shape=jax.ShapeDtypeStruct((_N, _K), jnp.float32),
    )(a, b, c)


def _add2(a, b):
    grid = _N // _BLK
    r = _row_spec(_BLK, _K)
    return pl.pallas_call(
        _add2_body,
        grid=(grid,),
        in_specs=[r, r],
        out_specs=r,
        out_shape=jax.ShapeDtypeStruct((_N, _K), jnp.float32),
    )(a, b)


def kernel(edge_index, Gu, Gi, feat_visual, Gum_visual, proj_W_visual, proj_b_visual, prop_W_visual_0, lin_W_visual_0, lin_b_visual_0, g_W_visual_0, g_b_visual_0, prop_W_visual_1, lin_W_visual_1, lin_b_visual_1, g_W_visual_1, g_b_visual_1, feat_textual, Gum_textual, proj_W_textual, proj_b_textual, prop_W_textual_0, lin_W_textual_0, lin_b_textual_0, g_W_textual_0, g_b_textual_0, prop_W_textual_1, lin_W_textual_1, lin_b_textual_1, g_W_textual_1, g_b_textual_1):
    npad = _EPAD - _E
    pad_src = jnp.arange(npad, dtype=jnp.int32) & 16383
    pad_dst = jnp.full((npad,), _N, jnp.int32) + (jnp.arange(npad, dtype=jnp.int32) & 15)
    src2d = jnp.concatenate([edge_index[0], pad_src]).reshape(_EPAD // _EROW, _EROW)
    dst2d = jnp.concatenate([edge_index[1], pad_dst]).reshape(_EPAD // _EROW, _EROW)
    zz = jnp.zeros((_ACC, _K), jnp.float32)

    csrc, cdst, counts = _edge_partition_sc(src2d, dst2d)
    csrc = csrc.reshape(2, _NSTRIP, _CROWS, _EROW)
    cdst = cdst.reshape(2, _NSTRIP, _CROWS, _EROW)

    bias0_v = (lin_b_visual_0 + g_b_visual_0).reshape(1, _K)
    bias1_v = (lin_b_visual_1 + g_b_visual_1).reshape(1, _K)
    bias0_t = (lin_b_textual_0 + g_b_textual_0).reshape(1, _K)
    bias1_t = (lin_b_textual_1 + g_b_textual_1).reshape(1, _K)

    z0_v, xh0_v, z0_t, xh0_t = _stage0(
        Gum_visual, feat_visual, proj_W_visual, proj_b_visual.reshape(1, _K),
        prop_W_visual_0, g_W_visual_0, lin_W_visual_0, bias0_v,
        Gum_textual, feat_textual, proj_W_textual, proj_b_textual.reshape(1, _K),
        prop_W_textual_0, g_W_textual_0, lin_W_textual_0, bias0_t, Gu, Gi)

    s0_v = _segment_sum_compact_sc(z0_v, csrc, cdst, counts, zz)
    s0_t = _segment_sum_compact_sc(z0_t, csrc, cdst, counts, zz)
    z1_v, xh1_v = _layer1(s0_v, xh0_v, prop_W_visual_1, g_W_visual_1,
                          lin_W_visual_1, bias1_v, Gu, Gi)

    s1_v = _segment_sum_compact_sc(z1_v, csrc, cdst, counts, zz)
    z1_t, xh1_t = _layer1(s0_t, xh0_t, prop_W_textual_1, g_W_textual_1,
                          lin_W_textual_1, bias1_t, Gu, Gi)

    s1_t = _segment_sum_compact_sc(z1_t, csrc, cdst, counts, zz)
    part = _add3(s1_v, xh1_v, xh1_t)

    x_all = _add2(part, s1_t)
    return x_all[:_NU], x_all[_NU:]
